# baseline (speedup 1.0000x reference)
import jax
import jax.numpy as jnp
from jax.experimental import pallas as pl

N = 50000
E = 800000
SCALAR = 32
VEC = 4
RBF_DIM = 10
RBF_DMAX = 32.0
LATENT = 8
VH = VEC + 1


def _norm_no_nan(x, axis=-1, eps=1e-8):
    return jnp.sqrt(jnp.sum(x * x, axis=axis) + eps)


def _rbf(d):
    mu = jnp.linspace(0.0, RBF_DMAX, RBF_DIM)
    sigma = RBF_DMAX / RBF_DIM
    return jnp.exp(-(((d[..., None] - mu) / sigma) ** 2))


def _mlp_kernel(s_ref, w1_ref, b1_ref, w2_ref, b2_ref, out_ref):
    h = jnp.maximum(s_ref[...] @ w1_ref[...] + b1_ref[...], 0.0)
    out_ref[...] = h @ w2_ref[...] + b2_ref[...]


def _final_mlp(s, W_l1, b_l1, W_l2, b_l2):
    n = s.shape[0]
    BN = 1000
    grid = n // BN
    return pl.pallas_call(
        _mlp_kernel,
        grid=(grid,),
        in_specs=[
            pl.BlockSpec((BN, SCALAR), lambda i: (i, 0)),
            pl.BlockSpec((SCALAR, 2 * LATENT), lambda i: (0, 0)),
            pl.BlockSpec((1, 2 * LATENT), lambda i: (0, 0)),
            pl.BlockSpec((2 * LATENT, LATENT), lambda i: (0, 0)),
            pl.BlockSpec((1, LATENT), lambda i: (0, 0)),
        ],
        out_specs=pl.BlockSpec((BN, LATENT), lambda i: (i, 0)),
        out_shape=jax.ShapeDtypeStruct((n, LATENT), jnp.float32),
    )(s, W_l1, b_l1[None, :], W_l2, b_l2[None, :])


def kernel(atom_types, atom_charges, bond_orders, coords, edge_index, a_table, c_table, e_table, W_ns, b_ns, Wh_0, Ws_0, bs_0, Wv_0, Wg_0, bg_0, Wh_1, Ws_1, bs_1, Wv_1, Wg_1, bg_1, Wh_2, Ws_2, bs_2, Wv_2, Wg_2, bg_2, W_l1, b_l1, W_l2, b_l2):
    n_nodes = atom_types.shape[0]
    s = jax.nn.relu(jnp.concatenate([a_table[atom_types], c_table[atom_charges]], axis=1) @ W_ns + b_ns)
    e_feat = e_table[bond_orders]
    v = jnp.zeros((n_nodes, VEC, 3), dtype=jnp.float32)
    src = edge_index[0]
    dst = edge_index[1]
    diff = coords[src] - coords[dst]
    d = _norm_no_nan(diff)
    x_diff = diff / d[:, None]
    d_rbf = _rbf(d)
    layers = [(Wh_0, Ws_0, bs_0, Wv_0, Wg_0, bg_0), (Wh_1, Ws_1, bs_1, Wv_1, Wg_1, bg_1), (Wh_2, Ws_2, bs_2, Wv_2, Wg_2, bg_2)]
    for Wh, Ws, bs, Wv, Wg, bg in layers:
        s_in = jnp.concatenate([s[src], e_feat, d_rbf], axis=1)
        v_in = jnp.concatenate([v[src], x_diff[:, None, :]], axis=1)
        vh = jnp.einsum("evc,vh->ehc", v_in, Wh)
        vn = _norm_no_nan(vh, axis=-1)
        ms = jax.nn.relu(jnp.concatenate([s_in, vn], axis=1) @ Ws + bs)
        mv = jnp.einsum("ehc,ho->eoc", vh, Wv)
        gate = jax.nn.sigmoid(ms @ Wg + bg)
        mv = mv * gate[:, :, None]
        agg_s = jax.ops.segment_sum(ms, dst, num_segments=n_nodes)
        agg_v = jax.ops.segment_sum(mv, dst, num_segments=n_nodes)
        cnt = jax.ops.segment_sum(jnp.ones((dst.shape[0],), dtype=jnp.float32), dst, num_segments=n_nodes)
        denom = jnp.maximum(cnt, 1.0)
        s = s + agg_s / denom[:, None]
        v = v + agg_v / denom[:, None, None]
    atom_latents = _final_mlp(s, W_l1, b_l1, W_l2, b_l2)
    mask = jnp.zeros((n_nodes,), dtype=bool)
    return (atom_latents, mask)


# speedup vs baseline: 24.1517x; 24.1517x over previous
import functools

import jax
import jax.numpy as jnp
from jax import lax
from jax.experimental import pallas as pl
from jax.experimental.pallas import tpu as pltpu
from jax.experimental.pallas import tpu_sc as plsc

N = 50000
E = 800000
SCALAR = 32
RBF_DIM = 10
RBF_DMAX = 32.0
LATENT = 8

EP = 819200
ROWS = EP // 128
NC = 2
NS = 16
NW = NC * NS
RPW = ROWS // NW
GK = 4
GSTEPS = RPW // GK
SK = 4
SSTEPS = RPW // SK

NP = 51200
NPS = NP // NS
PRS = NPS // 8
PROWS = NP // 8
ZCH = 1600
NCH = NPS // ZCH
PCH = ZCH // 8
NB = NPS
RB = 8
EB = RB * 128

_sigma = RBF_DMAX / RBF_DIM


def _mesh():
    return plsc.VectorSubcoreMesh(core_axis_name="c", subcore_axis_name="s",
                                  num_cores=NC, num_subcores=NS)


def _sc_gather(table, idx_r):
    @functools.partial(
        pl.kernel,
        out_type=jax.ShapeDtypeStruct((ROWS, 128, 128), jnp.float32),
        mesh=_mesh(),
        scratch_types=[pltpu.VMEM((GK, 128), jnp.int32),
                       pltpu.VMEM((GK, 128, 128), jnp.float32),
                       pltpu.SemaphoreType.DMA],
    )
    def k(tbl_h, idx_h, o_g, idxb, gbuf, sem):
        wid = lax.axis_index("s") * NC + lax.axis_index("c")

        def body(it, carry):
            rb = wid * RPW + it * GK
            pltpu.sync_copy(idx_h.at[pl.ds(rb, GK)], idxb)
            cps = [pltpu.async_copy(tbl_h.at[idxb.at[j]], gbuf.at[j], sem)
                   for j in range(GK)]
            for c in cps:
                c.wait()
            pltpu.sync_copy(gbuf, o_g.at[pl.ds(rb, GK)])
            return carry

        lax.fori_loop(0, GSTEPS, body, 0)

    return k(table, idx_r)


def _sc_scatter(vals, dst_r, nphase):
    @functools.partial(
        pl.kernel,
        out_type=jax.ShapeDtypeStruct((nphase, NC, PROWS, 128), jnp.float32),
        mesh=_mesh(),
        compiler_params=pltpu.CompilerParams(use_tc_tiling_on_sc=False),
        scratch_types=[pltpu.VMEM((SK, 128), jnp.int32),
                       pltpu.VMEM((SK, 128, 16), jnp.float32),
                       pltpu.VMEM((800, 16), jnp.float32),
                       pltpu.VMEM((ZCH, 16), jnp.float32),
                       pltpu.VMEM((PCH, 128), jnp.float32),
                       pltpu.VMEM_SHARED((NP, 16), jnp.float32)],
    )
    def k(vals_h, dst_h, out, idxb, vbuf, vz, vtmp, vstage, acc):
        cid = lax.axis_index("c")
        sid = lax.axis_index("s")

        def zb(i, carry):
            vz[i, :] = jnp.zeros((16,), jnp.float32)
            return carry

        lax.fori_loop(0, 800, zb, 0)

        for p in range(nphase):
            def zacc(i, carry):
                pltpu.sync_copy(vz, acc.at[pl.ds(sid * NPS + i * 800, 800)])
                return carry

            lax.fori_loop(0, 4, zacc, 0)
            plsc.subcore_barrier()

            def body(it, carry):
                rb = cid * (ROWS // NC) + sid * RPW + it * SK
                pltpu.sync_copy(dst_h.at[pl.ds(rb, SK)], idxb)
                pltpu.sync_copy(
                    vals_h.at[pl.ds(rb, SK), :, pl.ds(p * 16, 16)], vbuf)
                for j in range(SK):
                    pltpu.sync_copy(vbuf.at[j], acc.at[idxb.at[j]], add=True)
                return carry

            lax.fori_loop(0, SSTEPS, body, 0)
            plsc.subcore_barrier()

            for ch in range(NCH):
                pltpu.sync_copy(acc.at[pl.ds(sid * NPS + ch * ZCH, ZCH)], vtmp)
                for g in range(8):
                    def rp(r, carry):
                        vstage[r, pl.ds(g * 16, 16)] = vtmp[g * PCH + r, :]
                        return carry

                    lax.fori_loop(0, PCH, rp, 0)
                pltpu.sync_copy(
                    vstage, out.at[p, cid, pl.ds(sid * PRS + ch * PCH, PCH)])

    return _call_scatter(k, vals, dst_r)


def _call_scatter(k, vals, dst_r):
    return k(vals, dst_r)


def _unpack_partials(pp):
    parts = []
    for ch in range(NCH):
        for g in range(8):
            r0, r1 = ch * PCH, (ch + 1) * PCH
            c0, c1 = g * 16, (g + 1) * 16
            parts.append(pp[0, r0:r1, c0:c1] + pp[1, r0:r1, c0:c1])
    return jnp.concatenate(parts, axis=0)


def _embed_call(feat, A2, C2, b_ns):
    def body(f_ref, a_ref, c_ref, b_ref, o_ref):
        t = f_ref[:, 0:1]
        q = f_ref[:, 1:2]
        oha = (t == lax.broadcasted_iota(jnp.int32, (NB, 10), 1).astype(
            jnp.float32)).astype(jnp.float32)
        ohc = (q == lax.broadcasted_iota(jnp.int32, (NB, 6), 1).astype(
            jnp.float32)).astype(jnp.float32)
        s0 = jnp.maximum(oha @ a_ref[...] + ohc @ c_ref[...] + b_ref[...], 0.0)
        o_ref[...] = jnp.concatenate(
            [s0, jnp.zeros((NB, 12), jnp.float32), f_ref[:, 2:5],
             jnp.zeros((NB, 81), jnp.float32)], axis=1)

    return pl.pallas_call(
        body,
        grid=(NP // NB,),
        in_specs=[pl.BlockSpec((NB, 8), lambda i: (i, 0)),
                  pl.BlockSpec((10, SCALAR), lambda i: (0, 0)),
                  pl.BlockSpec((6, SCALAR), lambda i: (0, 0)),
                  pl.BlockSpec((1, SCALAR), lambda i: (0, 0))],
        out_specs=pl.BlockSpec((NB, 128), lambda i: (i, 0)),
        out_shape=jax.ShapeDtypeStruct((NP, 128), jnp.float32),
    )(feat, A2, C2, b_ns)


def _edge_core(ss, v15, d, bond, pid, wts, with_mv):
    (Ws_s, Tb, Ws_rbf, Ws_vn, bs, Wmat, Wv15, Wg, bg, E43) = wts
    vh = v15
    G = (lax.broadcasted_iota(jnp.int32, (15, 5), 0) // 3
         == lax.broadcasted_iota(jnp.int32, (15, 5), 1)).astype(jnp.float32)
    vn = jnp.sqrt((vh * vh) @ G + 1e-8)
    mu = lax.broadcasted_iota(jnp.int32, (EB, RBF_DIM), 1).astype(
        jnp.float32) * (RBF_DMAX / (RBF_DIM - 1))
    rbf = jnp.exp(-(((d - mu) / _sigma) ** 2))
    oh = (bond == lax.broadcasted_iota(jnp.int32, (EB, 5), 1).astype(
        jnp.float32)).astype(jnp.float32)
    pre = ss @ Ws_s + oh @ Tb + rbf @ Ws_rbf + vn @ Ws_vn + bs
    ms = jnp.maximum(pre, 0.0)
    row = pid * EB + lax.broadcasted_iota(jnp.int32, (EB, 1), 0)
    live = (row < E).astype(jnp.float32)
    ms = ms * live
    if not with_mv:
        return ms, None
    gate = jax.nn.sigmoid(ms @ Wg + bg)
    mv = (vh @ Wv15) * (gate @ E43)
    mvc = jnp.concatenate([mv * live, live, jnp.zeros((EB, 3), jnp.float32)],
                          axis=1)
    return ms, mvc


def _msg_block(ms, mvc):
    if mvc is None:
        mvc = jnp.zeros((EB, 16), jnp.float32)
    return jnp.concatenate([ms, mvc, jnp.zeros((EB, 80), jnp.float32)],
                           axis=1).reshape(RB, 128, 128)


def _geom_call(gs, gd, bond_r):
    def body(gs_ref, gd_ref, bd_ref, xd_ref):
        gs2 = gs_ref[...].reshape(EB, 128)
        gd2 = gd_ref[...].reshape(EB, 128)
        dif = gs2[:, 44:47] - gd2[:, 44:47]
        d = jnp.sqrt(jnp.sum(dif * dif, axis=1, keepdims=True) + 1e-8)
        x_diff = dif / d
        geo = jnp.concatenate([x_diff, d], axis=1)
        xd_ref[...] = jnp.concatenate(
            [geo.reshape(RB, 128, 4).swapaxes(1, 2), bd_ref[...][:, None, :],
             jnp.zeros((RB, 3, 128), jnp.float32)], axis=1)

    return pl.pallas_call(
        body,
        grid=(ROWS // RB,),
        in_specs=[pl.BlockSpec((RB, 128, 128), lambda i: (i, 0, 0)),
                  pl.BlockSpec((RB, 128, 128), lambda i: (i, 0, 0)),
                  pl.BlockSpec((RB, 128), lambda i: (i, 0))],
        out_specs=pl.BlockSpec((RB, 8, 128), lambda i: (i, 0, 0)),
        out_shape=jax.ShapeDtypeStruct((ROWS, 8, 128), jnp.float32),
    )(gs, gd, bond_r)


def _edgeN_call(g, xd, wts):
    (Ws_s, Tb, Ws_rbf, Ws_vn, bs, Wh15, Wv15, Wg, bg, E43) = wts

    def body(g_ref, x_ref, ws_ref, tb_ref, wr_ref, wv_ref, bs_ref, wh_ref,
             wv15_ref, wg_ref, bg_ref, e43_ref, msg_ref):
        g2 = g_ref[...].reshape(EB, 128)
        xt = x_ref[...].swapaxes(1, 2).reshape(EB, 8)
        xf = [xt[:, f:f + 1] for f in range(5)]
        d = xf[3]
        bond = xf[4]
        wh = wh_ref[...]
        vh = g2[:, 32:44] @ wh[0:12] + xf[0] * wh[12:13] + xf[1] * wh[13:14] \
            + xf[2] * wh[14:15]
        w = (ws_ref[...], tb_ref[...], wr_ref[...], wv_ref[...],
             bs_ref[...], None, wv15_ref[...], wg_ref[...], bg_ref[...],
             e43_ref[...])
        ms, mvc = _edge_core(g2[:, 0:32], vh, d, bond, pl.program_id(0), w,
                             True)
        msg_ref[...] = _msg_block(ms, mvc)

    wspecs = [pl.BlockSpec((32, 32), lambda i: (0, 0)),
              pl.BlockSpec((5, 32), lambda i: (0, 0)),
              pl.BlockSpec((10, 32), lambda i: (0, 0)),
              pl.BlockSpec((5, 32), lambda i: (0, 0)),
              pl.BlockSpec((1, 32), lambda i: (0, 0)),
              pl.BlockSpec((15, 15), lambda i: (0, 0)),
              pl.BlockSpec((15, 12), lambda i: (0, 0)),
              pl.BlockSpec((32, 4), lambda i: (0, 0)),
              pl.BlockSpec((1, 4), lambda i: (0, 0)),
              pl.BlockSpec((4, 12), lambda i: (0, 0))]

    return pl.pallas_call(
        body,
        grid=(ROWS // RB,),
        in_specs=[pl.BlockSpec((RB, 128, 128), lambda i: (i, 0, 0)),
                  pl.BlockSpec((RB, 8, 128), lambda i: (i, 0, 0))] + wspecs,
        out_specs=pl.BlockSpec((RB, 128, 128), lambda i: (i, 0, 0)),
        out_shape=jax.ShapeDtypeStruct((ROWS, 128, 128), jnp.float32),
    )(g, xd, Ws_s, Tb, Ws_rbf, Ws_vn, bs, Wh15, Wv15, Wg, bg, E43)


def _update_call(p, state):
    def body(p_ref, st_ref, o_ref):
        aggs = jnp.concatenate(
            [_unpack_partials(p_ref[0]), _unpack_partials(p_ref[1])], axis=1)
        aggv = _unpack_partials(p_ref[2])
        den = jnp.maximum(aggv[:, 12:13], 1.0)
        s1 = st_ref[:, 0:32] + aggs / den
        v1 = st_ref[:, 32:44] + aggv[:, 0:12] / den
        o_ref[...] = jnp.concatenate(
            [s1, v1, st_ref[:, 44:47], jnp.zeros((NB, 81), jnp.float32)],
            axis=1)

    return pl.pallas_call(
        body,
        grid=(NP // NB,),
        in_specs=[pl.BlockSpec((3, 2, PRS, 128), lambda i: (0, 0, i, 0)),
                  pl.BlockSpec((NB, 128), lambda i: (i, 0))],
        out_specs=pl.BlockSpec((NB, 128), lambda i: (i, 0)),
        out_shape=jax.ShapeDtypeStruct((NP, 128), jnp.float32),
    )(p, state)


def _mlp_call(state, W_l1, b_l1, W_l2, b_l2):
    def body(st_ref, w1_ref, b1_ref, w2_ref, b2_ref, o_ref):
        h = jnp.maximum(st_ref[:, 0:32] @ w1_ref[...] + b1_ref[...], 0.0)
        o_ref[...] = h @ w2_ref[...] + b2_ref[...]

    return pl.pallas_call(
        body,
        grid=(NP // NB,),
        in_specs=[pl.BlockSpec((NB, 128), lambda i: (i, 0)),
                  pl.BlockSpec((32, 16), lambda i: (0, 0)),
                  pl.BlockSpec((1, 16), lambda i: (0, 0)),
                  pl.BlockSpec((16, 8), lambda i: (0, 0)),
                  pl.BlockSpec((1, 8), lambda i: (0, 0))],
        out_specs=pl.BlockSpec((NB, LATENT), lambda i: (i, 0)),
        out_shape=jax.ShapeDtypeStruct((NP, LATENT), jnp.float32),
    )(state, W_l1, b_l1, W_l2, b_l2)


def _layer_weights(Wh, Ws, bs, Wv, Wg, bg, e_table, layer0):
    Ws_s = Ws[0:32]
    Tb = e_table @ Ws[32:40]
    Ws_rbf = Ws[40:50]
    Ws_vn = Ws[50:55]
    if layer0:
        Wmat = jnp.kron(Wh[4:5, :], jnp.eye(3, dtype=jnp.float32))
    else:
        Wmat = jnp.kron(Wh, jnp.eye(3, dtype=jnp.float32))
    Wv15 = jnp.kron(Wv, jnp.eye(3, dtype=jnp.float32))
    E43 = jnp.kron(jnp.eye(4, dtype=jnp.float32), jnp.ones((1, 3), jnp.float32))
    return (Ws_s, Tb, Ws_rbf, Ws_vn, bs[None, :], Wmat, Wv15, Wg, bg[None, :], E43)


def kernel(atom_types, atom_charges, bond_orders, coords, edge_index, a_table, c_table, e_table, W_ns, b_ns, Wh_0, Ws_0, bs_0, Wv_0, Wg_0, bg_0, Wh_1, Ws_1, bs_1, Wv_1, Wg_1, bg_1, Wh_2, Ws_2, bs_2, Wv_2, Wg_2, bg_2, W_l1, b_l1, W_l2, b_l2):
    pad = EP - E
    pad_idx = (jnp.arange(pad, dtype=jnp.int32) * 61) % N
    src_r = jnp.concatenate([edge_index[0].astype(jnp.int32), pad_idx]).reshape(ROWS, 128)
    dst_r = jnp.concatenate([edge_index[1].astype(jnp.int32), pad_idx]).reshape(ROWS, 128)
    bond_r = jnp.concatenate([bond_orders.astype(jnp.float32),
                              jnp.zeros((pad,), jnp.float32)]).reshape(ROWS, 128)
    feat = jnp.concatenate(
        [atom_types.astype(jnp.float32)[:, None],
         atom_charges.astype(jnp.float32)[:, None], coords,
         jnp.zeros((N, 3), jnp.float32)], axis=1)
    feat = jnp.concatenate([feat, jnp.zeros((NP - N, 8), jnp.float32)], axis=0)
    A2 = a_table @ W_ns[0:16]
    C2 = c_table @ W_ns[16:24]
    w0 = _layer_weights(Wh_0, Ws_0, bs_0, Wv_0, Wg_0, bg_0, e_table, False)
    w1 = _layer_weights(Wh_1, Ws_1, bs_1, Wv_1, Wg_1, bg_1, e_table, False)
    w2 = _layer_weights(Wh_2, Ws_2, bs_2, Wv_2, Wg_2, bg_2, e_table, False)

    state0 = _embed_call(feat, A2, C2, b_ns[None, :])

    gs0 = _sc_gather(state0, src_r)
    gd0 = _sc_gather(state0, dst_r)
    xd = _geom_call(gs0, gd0, bond_r)

    wstack = jax.tree.map(lambda *xs: jnp.stack(xs), w0, w1, w2)

    def layer(state, wl):
        g = _sc_gather(state, src_r)
        msg = _edgeN_call(g, xd, wl)
        p = _sc_scatter(msg, dst_r, 3)
        return _update_call(p, state), None

    state3, _ = lax.scan(layer, state0, wstack)

    lat = _mlp_call(state3, W_l1, b_l1[None, :], W_l2, b_l2[None, :])

    atom_latents = lat[:N]
    mask = jnp.zeros((N,), dtype=bool)
    return (atom_latents, mask)


# speedup vs baseline: 26.8481x; 1.1116x over previous
import functools

import jax
import jax.numpy as jnp
from jax import lax
from jax.experimental import pallas as pl
from jax.experimental.pallas import tpu as pltpu
from jax.experimental.pallas import tpu_sc as plsc

N = 50000
E = 800000
SCALAR = 32
RBF_DIM = 10
RBF_DMAX = 32.0
LATENT = 8

EP = 819200
ROWS = EP // 128
NC = 2
NS = 16
NW = NC * NS
RPW = ROWS // NW
GK = 4
GSTEPS = RPW // GK
SK = 4
SSTEPS = RPW // SK

NP = 51200
NPS = NP // NS
PRS = NPS // 8
PROWS = NP // 8
ZCH = 1600
NCH = NPS // ZCH
PCH = ZCH // 8
NB = NPS
RB = 16
EB = RB * 128

_sigma = RBF_DMAX / RBF_DIM


def _mesh():
    return plsc.VectorSubcoreMesh(core_axis_name="c", subcore_axis_name="s",
                                  num_cores=NC, num_subcores=NS)


def _sc_gather(table, idx_r):
    @functools.partial(
        pl.kernel,
        out_type=jax.ShapeDtypeStruct((ROWS, 128, 128), jnp.float32),
        mesh=_mesh(),
        scratch_types=[pltpu.VMEM((GK, 128), jnp.int32),
                       pltpu.VMEM((GK, 128, 128), jnp.float32),
                       pltpu.SemaphoreType.DMA],
    )
    def k(tbl_h, idx_h, o_g, idxb, gbuf, sem):
        wid = lax.axis_index("s") * NC + lax.axis_index("c")

        def body(it, carry):
            rb = wid * RPW + it * GK
            pltpu.sync_copy(idx_h.at[pl.ds(rb, GK)], idxb)
            cps = [pltpu.async_copy(tbl_h.at[idxb.at[j]], gbuf.at[j], sem)
                   for j in range(GK)]
            for c in cps:
                c.wait()
            pltpu.sync_copy(gbuf, o_g.at[pl.ds(rb, GK)])
            return carry

        lax.fori_loop(0, GSTEPS, body, 0)

    return k(table, idx_r)


def _sc_scatter(vals, dst_r, nphase):
    @functools.partial(
        pl.kernel,
        out_type=jax.ShapeDtypeStruct((nphase, NC, PROWS, 128), jnp.float32),
        mesh=_mesh(),
        compiler_params=pltpu.CompilerParams(use_tc_tiling_on_sc=False),
        scratch_types=[pltpu.VMEM((SK, 128), jnp.int32),
                       pltpu.VMEM((SK, 128, 16), jnp.float32),
                       pltpu.VMEM((800, 16), jnp.float32),
                       pltpu.VMEM((ZCH, 16), jnp.float32),
                       pltpu.VMEM((PCH, 128), jnp.float32),
                       pltpu.VMEM_SHARED((NP, 16), jnp.float32)],
    )
    def k(vals_h, dst_h, out, idxb, vbuf, vz, vtmp, vstage, acc):
        cid = lax.axis_index("c")
        sid = lax.axis_index("s")

        def zb(i, carry):
            vz[i, :] = jnp.zeros((16,), jnp.float32)
            return carry

        lax.fori_loop(0, 800, zb, 0)

        for p in range(nphase):
            def zacc(i, carry):
                pltpu.sync_copy(vz, acc.at[pl.ds(sid * NPS + i * 800, 800)])
                return carry

            lax.fori_loop(0, 4, zacc, 0)
            plsc.subcore_barrier()

            def body(it, carry):
                rb = cid * (ROWS // NC) + sid * RPW + it * SK
                pltpu.sync_copy(dst_h.at[pl.ds(rb, SK)], idxb)
                pltpu.sync_copy(
                    vals_h.at[pl.ds(rb, SK), :, pl.ds(p * 16, 16)], vbuf)
                for j in range(SK):
                    pltpu.sync_copy(vbuf.at[j], acc.at[idxb.at[j]], add=True)
                return carry

            lax.fori_loop(0, SSTEPS, body, 0)
            plsc.subcore_barrier()

            for ch in range(NCH):
                pltpu.sync_copy(acc.at[pl.ds(sid * NPS + ch * ZCH, ZCH)], vtmp)
                for g in range(8):
                    def rp(r, carry):
                        vstage[r, pl.ds(g * 16, 16)] = vtmp[g * PCH + r, :]
                        return carry

                    lax.fori_loop(0, PCH, rp, 0)
                pltpu.sync_copy(
                    vstage, out.at[p, cid, pl.ds(sid * PRS + ch * PCH, PCH)])

    return _call_scatter(k, vals, dst_r)


def _call_scatter(k, vals, dst_r):
    return k(vals, dst_r)


def _unpack_partials(pp):
    parts = []
    for ch in range(NCH):
        for g in range(8):
            r0, r1 = ch * PCH, (ch + 1) * PCH
            c0, c1 = g * 16, (g + 1) * 16
            parts.append(pp[0, r0:r1, c0:c1] + pp[1, r0:r1, c0:c1])
    return jnp.concatenate(parts, axis=0)


def _embed_call(feat, A2, C2, b_ns):
    def body(f_ref, a_ref, c_ref, b_ref, o_ref):
        t = f_ref[:, 0:1]
        q = f_ref[:, 1:2]
        oha = (t == lax.broadcasted_iota(jnp.int32, (NB, 10), 1).astype(
            jnp.float32)).astype(jnp.float32)
        ohc = (q == lax.broadcasted_iota(jnp.int32, (NB, 6), 1).astype(
            jnp.float32)).astype(jnp.float32)
        s0 = jnp.maximum(oha @ a_ref[...] + ohc @ c_ref[...] + b_ref[...], 0.0)
        o_ref[...] = jnp.concatenate(
            [s0, jnp.zeros((NB, 12), jnp.float32), f_ref[:, 2:5],
             jnp.zeros((NB, 81), jnp.float32)], axis=1)

    return pl.pallas_call(
        body,
        grid=(NP // NB,),
        in_specs=[pl.BlockSpec((NB, 8), lambda i: (i, 0)),
                  pl.BlockSpec((10, SCALAR), lambda i: (0, 0)),
                  pl.BlockSpec((6, SCALAR), lambda i: (0, 0)),
                  pl.BlockSpec((1, SCALAR), lambda i: (0, 0))],
        out_specs=pl.BlockSpec((NB, 128), lambda i: (i, 0)),
        out_shape=jax.ShapeDtypeStruct((NP, 128), jnp.float32),
    )(feat, A2, C2, b_ns)


def _edge_core(ss, v15, d, bond, pid, wts, with_mv):
    (Ws_s, Tb, Ws_rbf, Ws_vn, bs, Wmat, Wv15, Wg, bg, E43) = wts
    vh = v15
    G = (lax.broadcasted_iota(jnp.int32, (15, 5), 0) // 3
         == lax.broadcasted_iota(jnp.int32, (15, 5), 1)).astype(jnp.float32)
    vn = jnp.sqrt((vh * vh) @ G + 1e-8)
    mu = lax.broadcasted_iota(jnp.int32, (EB, RBF_DIM), 1).astype(
        jnp.float32) * (RBF_DMAX / (RBF_DIM - 1))
    rbf = jnp.exp(-(((d - mu) / _sigma) ** 2))
    oh = (bond == lax.broadcasted_iota(jnp.int32, (EB, 5), 1).astype(
        jnp.float32)).astype(jnp.float32)
    pre = ss @ Ws_s + oh @ Tb + rbf @ Ws_rbf + vn @ Ws_vn + bs
    ms = jnp.maximum(pre, 0.0)
    row = pid * EB + lax.broadcasted_iota(jnp.int32, (EB, 1), 0)
    live = (row < E).astype(jnp.float32)
    ms = ms * live
    if not with_mv:
        return ms, None
    gate = jax.nn.sigmoid(ms @ Wg + bg)
    mv = (vh @ Wv15) * (gate @ E43)
    mvc = jnp.concatenate([mv * live, live, jnp.zeros((EB, 3), jnp.float32)],
                          axis=1)
    return ms, mvc


def _msg_block(ms, mvc):
    if mvc is None:
        mvc = jnp.zeros((EB, 16), jnp.float32)
    return jnp.concatenate([ms, mvc, jnp.zeros((EB, 80), jnp.float32)],
                           axis=1).reshape(RB, 128, 128)


def _geom_call(gs, gd, bond_r):
    def body(gs_ref, gd_ref, bd_ref, msg_ref):
        gs2 = gs_ref[...].reshape(EB, 128)
        gd2 = gd_ref[...].reshape(EB, 128)
        dif = gs2[:, 44:47] - gd2[:, 44:47]
        d = jnp.sqrt(jnp.sum(dif * dif, axis=1, keepdims=True) + 1e-8)
        x_diff = dif / d
        geo = jnp.concatenate([x_diff, d], axis=1).reshape(RB, 128, 4)
        msg_ref[...] = jnp.concatenate(
            [jnp.zeros((RB, 128, 48), jnp.float32), geo,
             bd_ref[...][:, :, None], jnp.zeros((RB, 128, 75), jnp.float32)],
            axis=2)

    return pl.pallas_call(
        body,
        grid=(ROWS // RB,),
        in_specs=[pl.BlockSpec((RB, 128, 128), lambda i: (i, 0, 0)),
                  pl.BlockSpec((RB, 128, 128), lambda i: (i, 0, 0)),
                  pl.BlockSpec((RB, 128), lambda i: (i, 0))],
        out_specs=pl.BlockSpec((RB, 128, 128), lambda i: (i, 0, 0)),
        out_shape=jax.ShapeDtypeStruct((ROWS, 128, 128), jnp.float32),
    )(gs, gd, bond_r)


def _edgeN_call(g, msg_prev, wts):
    (Ws_s, Tb, Ws_rbf, Ws_vn, bs, Wh15, Wv15, Wg, bg, E43) = wts

    def body(g_ref, m_ref, ws_ref, tb_ref, wr_ref, wv_ref, bs_ref, wh_ref,
             wv15_ref, wg_ref, bg_ref, e43_ref, msg_ref):
        g2 = g_ref[...].reshape(EB, 128)
        m2 = m_ref[...].reshape(EB, 128)
        x_diff = m2[:, 48:51]
        d = m2[:, 51:52]
        bond = m2[:, 52:53]
        wh = wh_ref[...]
        vh = g2[:, 32:44] @ wh[0:12] + x_diff @ wh[12:15]
        w = (ws_ref[...], tb_ref[...], wr_ref[...], wv_ref[...],
             bs_ref[...], None, wv15_ref[...], wg_ref[...], bg_ref[...],
             e43_ref[...])
        ms, mvc = _edge_core(g2[:, 0:32], vh, d, bond, pl.program_id(0), w,
                             True)
        msg_ref[...] = jnp.concatenate(
            [ms, mvc, m2[:, 48:56], jnp.zeros((EB, 72), jnp.float32)],
            axis=1).reshape(RB, 128, 128)

    wspecs = [pl.BlockSpec((32, 32), lambda i: (0, 0)),
              pl.BlockSpec((5, 32), lambda i: (0, 0)),
              pl.BlockSpec((10, 32), lambda i: (0, 0)),
              pl.BlockSpec((5, 32), lambda i: (0, 0)),
              pl.BlockSpec((1, 32), lambda i: (0, 0)),
              pl.BlockSpec((15, 15), lambda i: (0, 0)),
              pl.BlockSpec((15, 12), lambda i: (0, 0)),
              pl.BlockSpec((32, 4), lambda i: (0, 0)),
              pl.BlockSpec((1, 4), lambda i: (0, 0)),
              pl.BlockSpec((4, 12), lambda i: (0, 0))]

    return pl.pallas_call(
        body,
        grid=(ROWS // RB,),
        in_specs=[pl.BlockSpec((RB, 128, 128), lambda i: (i, 0, 0)),
                  pl.BlockSpec((RB, 128, 128), lambda i: (i, 0, 0))] + wspecs,
        out_specs=pl.BlockSpec((RB, 128, 128), lambda i: (i, 0, 0)),
        out_shape=jax.ShapeDtypeStruct((ROWS, 128, 128), jnp.float32),
    )(g, msg_prev, Ws_s, Tb, Ws_rbf, Ws_vn, bs, Wh15, Wv15, Wg, bg, E43)


def _update_call(p, state):
    def body(p_ref, st_ref, o_ref):
        aggs = jnp.concatenate(
            [_unpack_partials(p_ref[0]), _unpack_partials(p_ref[1])], axis=1)
        aggv = _unpack_partials(p_ref[2])
        den = jnp.maximum(aggv[:, 12:13], 1.0)
        s1 = st_ref[:, 0:32] + aggs / den
        v1 = st_ref[:, 32:44] + aggv[:, 0:12] / den
        o_ref[...] = jnp.concatenate(
            [s1, v1, st_ref[:, 44:47], jnp.zeros((NB, 81), jnp.float32)],
            axis=1)

    return pl.pallas_call(
        body,
        grid=(NP // NB,),
        in_specs=[pl.BlockSpec((3, 2, PRS, 128), lambda i: (0, 0, i, 0)),
                  pl.BlockSpec((NB, 128), lambda i: (i, 0))],
        out_specs=pl.BlockSpec((NB, 128), lambda i: (i, 0)),
        out_shape=jax.ShapeDtypeStruct((NP, 128), jnp.float32),
    )(p, state)


def _mlp_call(state, W_l1, b_l1, W_l2, b_l2):
    def body(st_ref, w1_ref, b1_ref, w2_ref, b2_ref, o_ref):
        h = jnp.maximum(st_ref[:, 0:32] @ w1_ref[...] + b1_ref[...], 0.0)
        o_ref[...] = h @ w2_ref[...] + b2_ref[...]

    return pl.pallas_call(
        body,
        grid=(NP // NB,),
        in_specs=[pl.BlockSpec((NB, 128), lambda i: (i, 0)),
                  pl.BlockSpec((32, 16), lambda i: (0, 0)),
                  pl.BlockSpec((1, 16), lambda i: (0, 0)),
                  pl.BlockSpec((16, 8), lambda i: (0, 0)),
                  pl.BlockSpec((1, 8), lambda i: (0, 0))],
        out_specs=pl.BlockSpec((NB, LATENT), lambda i: (i, 0)),
        out_shape=jax.ShapeDtypeStruct((NP, LATENT), jnp.float32),
    )(state, W_l1, b_l1, W_l2, b_l2)


def _layer_weights(Wh, Ws, bs, Wv, Wg, bg, e_table, layer0):
    Ws_s = Ws[0:32]
    Tb = e_table @ Ws[32:40]
    Ws_rbf = Ws[40:50]
    Ws_vn = Ws[50:55]
    if layer0:
        Wmat = jnp.kron(Wh[4:5, :], jnp.eye(3, dtype=jnp.float32))
    else:
        Wmat = jnp.kron(Wh, jnp.eye(3, dtype=jnp.float32))
    Wv15 = jnp.kron(Wv, jnp.eye(3, dtype=jnp.float32))
    E43 = jnp.kron(jnp.eye(4, dtype=jnp.float32), jnp.ones((1, 3), jnp.float32))
    return (Ws_s, Tb, Ws_rbf, Ws_vn, bs[None, :], Wmat, Wv15, Wg, bg[None, :], E43)


def kernel(atom_types, atom_charges, bond_orders, coords, edge_index, a_table, c_table, e_table, W_ns, b_ns, Wh_0, Ws_0, bs_0, Wv_0, Wg_0, bg_0, Wh_1, Ws_1, bs_1, Wv_1, Wg_1, bg_1, Wh_2, Ws_2, bs_2, Wv_2, Wg_2, bg_2, W_l1, b_l1, W_l2, b_l2):
    pad = EP - E
    pad_idx = (jnp.arange(pad, dtype=jnp.int32) * 61) % N
    src_r = jnp.concatenate([edge_index[0].astype(jnp.int32), pad_idx]).reshape(ROWS, 128)
    dst_r = jnp.concatenate([edge_index[1].astype(jnp.int32), pad_idx]).reshape(ROWS, 128)
    bond_r = jnp.concatenate([bond_orders.astype(jnp.float32),
                              jnp.zeros((pad,), jnp.float32)]).reshape(ROWS, 128)
    feat = jnp.concatenate(
        [atom_types.astype(jnp.float32)[:, None],
         atom_charges.astype(jnp.float32)[:, None], coords,
         jnp.zeros((N, 3), jnp.float32)], axis=1)
    feat = jnp.concatenate([feat, jnp.zeros((NP - N, 8), jnp.float32)], axis=0)
    A2 = a_table @ W_ns[0:16]
    C2 = c_table @ W_ns[16:24]
    w0 = _layer_weights(Wh_0, Ws_0, bs_0, Wv_0, Wg_0, bg_0, e_table, False)
    w1 = _layer_weights(Wh_1, Ws_1, bs_1, Wv_1, Wg_1, bg_1, e_table, False)
    w2 = _layer_weights(Wh_2, Ws_2, bs_2, Wv_2, Wg_2, bg_2, e_table, False)

    state0 = _embed_call(feat, A2, C2, b_ns[None, :])

    gs0 = _sc_gather(state0, src_r)
    gd0 = _sc_gather(state0, dst_r)
    msg_init = _geom_call(gs0, gd0, bond_r)

    wstack = jax.tree.map(lambda *xs: jnp.stack(xs), w0, w1, w2)

    def layer(carry, wl):
        state, msg_prev = carry
        g = _sc_gather(state, src_r)
        msg = _edgeN_call(g, msg_prev, wl)
        p = _sc_scatter(msg, dst_r, 3)
        return (_update_call(p, state), msg), None

    (state3, _), _ = lax.scan(layer, (state0, msg_init), wstack)

    lat = _mlp_call(state3, W_l1, b_l1[None, :], W_l2, b_l2[None, :])

    atom_latents = lat[:N]
    mask = jnp.zeros((N,), dtype=bool)
    return (atom_latents, mask)


# speedup vs baseline: 32.8960x; 1.2253x over previous
import functools

import jax
import jax.numpy as jnp
from jax import lax
from jax.experimental import pallas as pl
from jax.experimental.pallas import tpu as pltpu
from jax.experimental.pallas import tpu_sc as plsc

N = 50000
E = 800000
SCALAR = 32
RBF_DIM = 10
RBF_DMAX = 32.0
LATENT = 8

EP = 819200
ROWS = EP // 128
NC = 2
NS = 16
NW = NC * NS
RPW = ROWS // NW
GK = 2
GSTEPS = RPW // GK
SK = 8
SSTEPS = RPW // SK

NP = 51200
NPS = NP // NS
PRS = NPS // 8
PROWS = NP // 8
ZCH = 640
NCH = NPS // ZCH
PCH = ZCH // 8
NB = NPS
RB = 16
EB = RB * 128

_sigma = RBF_DMAX / RBF_DIM


def _mesh():
    return plsc.VectorSubcoreMesh(core_axis_name="c", subcore_axis_name="s",
                                  num_cores=NC, num_subcores=NS)


def _sc_gather(table, idx_r, W):
    @functools.partial(
        pl.kernel,
        out_type=jax.ShapeDtypeStruct((ROWS, 128, 128), jnp.float32),
        mesh=_mesh(),
        compiler_params=pltpu.CompilerParams(use_tc_tiling_on_sc=False),
        scratch_types=[pltpu.VMEM((2, GK, 128), jnp.int32),
                       pltpu.VMEM((2, GK, 128, W), jnp.float32),
                       pltpu.SemaphoreType.DMA,
                       pltpu.SemaphoreType.DMA,
                       pltpu.SemaphoreType.DMA,
                       pltpu.SemaphoreType.DMA],
    )
    def k(tbl_h, idx_h, o_g, idxb, gbuf, gs0, gs1, os0, os1):
        wid = lax.axis_index("s") * NC + lax.axis_index("c")
        gsems = (gs0, gs1)
        osems = (os0, os1)

        def fire(s, b):
            rb = wid * RPW + s * GK
            pltpu.sync_copy(idx_h.at[pl.ds(rb, GK)], idxb.at[b])
            for j in range(GK):
                pltpu.async_copy(tbl_h.at[idxb.at[b, j]], gbuf.at[b, j],
                                 gsems[b])

        def drain_g(b):
            for j in range(GK):
                pltpu.make_async_copy(tbl_h.at[idxb.at[b, j]], gbuf.at[b, j],
                                      gsems[b]).wait()

        def flush(s, b):
            rb = wid * RPW + s * GK
            pltpu.async_copy(gbuf.at[b], o_g.at[pl.ds(rb, GK), :, pl.ds(0, W)],
                             osems[b])

        def drain_o(s, b):
            rb = wid * RPW + s * GK
            pltpu.make_async_copy(gbuf.at[b],
                                  o_g.at[pl.ds(rb, GK), :, pl.ds(0, W)],
                                  osems[b]).wait()

        fire(0, 0)

        def body(it, carry):
            for ph in range(2):
                s = it * 2 + ph
                b = ph
                nb = 1 - ph

                @pl.when(s >= 1)
                def _():
                    drain_o(s - 1, nb)

                @pl.when(s + 1 < GSTEPS)
                def _():
                    fire(s + 1, nb)

                drain_g(b)
                flush(s, b)
            return carry

        lax.fori_loop(0, GSTEPS // 2, body, 0)
        drain_o(GSTEPS - 1, (GSTEPS - 1) % 2)

    return k(table, idx_r)


def _sc_scatter(vals, dst_r, nphase):
    @functools.partial(
        pl.kernel,
        out_type=jax.ShapeDtypeStruct((nphase, NC, PROWS, 128), jnp.float32),
        mesh=_mesh(),
        compiler_params=pltpu.CompilerParams(use_tc_tiling_on_sc=False),
        scratch_types=[pltpu.VMEM((2, SK, 128), jnp.int32),
                       pltpu.VMEM((2, SK, 128, 16), jnp.float32),
                       pltpu.VMEM((ZCH, 16), jnp.float32),
                       pltpu.VMEM((ZCH, 16), jnp.float32),
                       pltpu.VMEM((PCH, 128), jnp.float32),
                       pltpu.SemaphoreType.DMA,
                       pltpu.SemaphoreType.DMA,
                       pltpu.VMEM_SHARED((NP, 16), jnp.float32)],
    )
    def k(vals_h, dst_h, out, idxb, vbuf, vz, vtmp, vstage, ls0, ls1, acc):
        cid = lax.axis_index("c")
        sid = lax.axis_index("s")
        lsems = (ls0, ls1)

        def zb(i, carry):
            vz[i, :] = jnp.zeros((16,), jnp.float32)
            return carry

        lax.fori_loop(0, ZCH, zb, 0)

        for p in range(nphase):
            def zacc(i, carry):
                pltpu.sync_copy(vz, acc.at[pl.ds(sid * NPS + i * ZCH, ZCH)])
                return carry

            lax.fori_loop(0, NCH, zacc, 0)
            plsc.subcore_barrier()

            def fire(s, b):
                rb = cid * (ROWS // NC) + sid * RPW + s * SK
                pltpu.sync_copy(dst_h.at[pl.ds(rb, SK)], idxb.at[b])
                pltpu.async_copy(
                    vals_h.at[pl.ds(rb, SK), :, pl.ds(p * 16, 16)],
                    vbuf.at[b], lsems[b])

            def drain(s, b):
                rb = cid * (ROWS // NC) + sid * RPW + s * SK
                pltpu.make_async_copy(
                    vals_h.at[pl.ds(rb, SK), :, pl.ds(p * 16, 16)],
                    vbuf.at[b], lsems[b]).wait()

            fire(0, 0)

            def body(it, carry):
                for ph in range(2):
                    s = it * 2 + ph
                    b = ph

                    @pl.when(s < SSTEPS)
                    def _():
                        @pl.when(s + 1 < SSTEPS)
                        def _():
                            fire(s + 1, 1 - ph)

                        drain(s, b)
                        for j in range(SK):
                            pltpu.sync_copy(vbuf.at[b, j],
                                            acc.at[idxb.at[b, j]], add=True)
                return carry

            lax.fori_loop(0, (SSTEPS + 1) // 2, body, 0)
            plsc.subcore_barrier()

            for ch in range(NCH):
                pltpu.sync_copy(acc.at[pl.ds(sid * NPS + ch * ZCH, ZCH)], vtmp)
                for g in range(8):
                    def rp(r, carry):
                        vstage[r, pl.ds(g * 16, 16)] = vtmp[g * PCH + r, :]
                        return carry

                    lax.fori_loop(0, PCH, rp, 0)
                pltpu.sync_copy(
                    vstage, out.at[p, cid, pl.ds(sid * PRS + ch * PCH, PCH)])

    return _call_scatter(k, vals, dst_r)


def _call_scatter(k, vals, dst_r):
    return k(vals, dst_r)


def _unpack_partials(pp):
    parts = []
    for ch in range(NCH):
        for g in range(8):
            r0, r1 = ch * PCH, (ch + 1) * PCH
            c0, c1 = g * 16, (g + 1) * 16
            parts.append(pp[0, r0:r1, c0:c1] + pp[1, r0:r1, c0:c1])
    return jnp.concatenate(parts, axis=0)


def _embed_call(feat, A2, C2, b_ns):
    def body(f_ref, a_ref, c_ref, b_ref, o_ref):
        t = f_ref[:, 0:1]
        q = f_ref[:, 1:2]
        oha = (t == lax.broadcasted_iota(jnp.int32, (NB, 10), 1).astype(
            jnp.float32)).astype(jnp.float32)
        ohc = (q == lax.broadcasted_iota(jnp.int32, (NB, 6), 1).astype(
            jnp.float32)).astype(jnp.float32)
        s0 = jnp.maximum(oha @ a_ref[...] + ohc @ c_ref[...] + b_ref[...], 0.0)
        o_ref[...] = jnp.concatenate(
            [s0, jnp.zeros((NB, 12), jnp.float32), f_ref[:, 2:5],
             jnp.zeros((NB, 81), jnp.float32)], axis=1)

    return pl.pallas_call(
        body,
        grid=(NP // NB,),
        in_specs=[pl.BlockSpec((NB, 8), lambda i: (i, 0)),
                  pl.BlockSpec((10, SCALAR), lambda i: (0, 0)),
                  pl.BlockSpec((6, SCALAR), lambda i: (0, 0)),
                  pl.BlockSpec((1, SCALAR), lambda i: (0, 0))],
        out_specs=pl.BlockSpec((NB, 128), lambda i: (i, 0)),
        out_shape=jax.ShapeDtypeStruct((NP, 128), jnp.float32),
    )(feat, A2, C2, b_ns)


def _edge_core(ss, v15, d, bond, pid, wts, with_mv):
    (Ws_s, Tb, Ws_rbf, Ws_vn, bs, Wmat, Wv15, Wg, bg, E43) = wts
    vh = v15
    G = (lax.broadcasted_iota(jnp.int32, (15, 5), 0) // 3
         == lax.broadcasted_iota(jnp.int32, (15, 5), 1)).astype(jnp.float32)
    vn = jnp.sqrt((vh * vh) @ G + 1e-8)
    mu = lax.broadcasted_iota(jnp.int32, (EB, RBF_DIM), 1).astype(
        jnp.float32) * (RBF_DMAX / (RBF_DIM - 1))
    rbf = jnp.exp(-(((d - mu) / _sigma) ** 2))
    oh = (bond == lax.broadcasted_iota(jnp.int32, (EB, 5), 1).astype(
        jnp.float32)).astype(jnp.float32)
    pre = ss @ Ws_s + oh @ Tb + rbf @ Ws_rbf + vn @ Ws_vn + bs
    ms = jnp.maximum(pre, 0.0)
    row = pid * EB + lax.broadcasted_iota(jnp.int32, (EB, 1), 0)
    live = (row < E).astype(jnp.float32)
    ms = ms * live
    if not with_mv:
        return ms, None
    gate = jax.nn.sigmoid(ms @ Wg + bg)
    mv = (vh @ Wv15) * (gate @ E43)
    mvc = jnp.concatenate([mv * live, live, jnp.zeros((EB, 3), jnp.float32)],
                          axis=1)
    return ms, mvc


def _msg_block(ms, mvc):
    if mvc is None:
        mvc = jnp.zeros((EB, 16), jnp.float32)
    return jnp.concatenate([ms, mvc, jnp.zeros((EB, 80), jnp.float32)],
                           axis=1).reshape(RB, 128, 128)


def _geom_call(gs, gd, bond_r):
    def body(gs_ref, gd_ref, bd_ref, msg_ref):
        gs2 = gs_ref[...].reshape(EB, 128)
        gd2 = gd_ref[...].reshape(EB, 128)
        dif = gs2[:, 0:3] - gd2[:, 0:3]
        d = jnp.sqrt(jnp.sum(dif * dif, axis=1, keepdims=True) + 1e-8)
        x_diff = dif / d
        geo = jnp.concatenate([x_diff, d], axis=1).reshape(RB, 128, 4)
        msg_ref[...] = jnp.concatenate(
            [jnp.zeros((RB, 128, 48), jnp.float32), geo,
             bd_ref[...][:, :, None], jnp.zeros((RB, 128, 75), jnp.float32)],
            axis=2)

    return pl.pallas_call(
        body,
        grid=(ROWS // RB,),
        in_specs=[pl.BlockSpec((RB, 128, 128), lambda i: (i, 0, 0)),
                  pl.BlockSpec((RB, 128, 128), lambda i: (i, 0, 0)),
                  pl.BlockSpec((RB, 128), lambda i: (i, 0))],
        out_specs=pl.BlockSpec((RB, 128, 128), lambda i: (i, 0, 0)),
        out_shape=jax.ShapeDtypeStruct((ROWS, 128, 128), jnp.float32),
    )(gs, gd, bond_r)


def _edgeN_call(g, msg_prev, wts):
    (Ws_s, Tb, Ws_rbf, Ws_vn, bs, Wh15, Wv15, Wg, bg, E43) = wts

    def body(g_ref, m_ref, ws_ref, tb_ref, wr_ref, wv_ref, bs_ref, wh_ref,
             wv15_ref, wg_ref, bg_ref, e43_ref, msg_ref):
        g2 = g_ref[...].reshape(EB, 128)
        m2 = m_ref[...].reshape(EB, 128)
        x_diff = m2[:, 48:51]
        d = m2[:, 51:52]
        bond = m2[:, 52:53]
        wh = wh_ref[...]
        vh = g2[:, 32:44] @ wh[0:12] + x_diff @ wh[12:15]
        w = (ws_ref[...], tb_ref[...], wr_ref[...], wv_ref[...],
             bs_ref[...], None, wv15_ref[...], wg_ref[...], bg_ref[...],
             e43_ref[...])
        ms, mvc = _edge_core(g2[:, 0:32], vh, d, bond, pl.program_id(0), w,
                             True)
        msg_ref[...] = jnp.concatenate(
            [ms, mvc, m2[:, 48:56], jnp.zeros((EB, 72), jnp.float32)],
            axis=1).reshape(RB, 128, 128)

    wspecs = [pl.BlockSpec((32, 32), lambda i: (0, 0)),
              pl.BlockSpec((5, 32), lambda i: (0, 0)),
              pl.BlockSpec((10, 32), lambda i: (0, 0)),
              pl.BlockSpec((5, 32), lambda i: (0, 0)),
              pl.BlockSpec((1, 32), lambda i: (0, 0)),
              pl.BlockSpec((15, 15), lambda i: (0, 0)),
              pl.BlockSpec((15, 12), lambda i: (0, 0)),
              pl.BlockSpec((32, 4), lambda i: (0, 0)),
              pl.BlockSpec((1, 4), lambda i: (0, 0)),
              pl.BlockSpec((4, 12), lambda i: (0, 0))]

    return pl.pallas_call(
        body,
        grid=(ROWS // RB,),
        in_specs=[pl.BlockSpec((RB, 128, 128), lambda i: (i, 0, 0)),
                  pl.BlockSpec((RB, 128, 128), lambda i: (i, 0, 0))] + wspecs,
        out_specs=pl.BlockSpec((RB, 128, 128), lambda i: (i, 0, 0)),
        out_shape=jax.ShapeDtypeStruct((ROWS, 128, 128), jnp.float32),
    )(g, msg_prev, Ws_s, Tb, Ws_rbf, Ws_vn, bs, Wh15, Wv15, Wg, bg, E43)


def _update_call(p, state):
    def body(p_ref, st_ref, o_ref):
        aggs = jnp.concatenate(
            [_unpack_partials(p_ref[0]), _unpack_partials(p_ref[1])], axis=1)
        aggv = _unpack_partials(p_ref[2])
        den = jnp.maximum(aggv[:, 12:13], 1.0)
        s1 = st_ref[:, 0:32] + aggs / den
        v1 = st_ref[:, 32:44] + aggv[:, 0:12] / den
        o_ref[...] = jnp.concatenate(
            [s1, v1, st_ref[:, 44:47], jnp.zeros((NB, 81), jnp.float32)],
            axis=1)

    return pl.pallas_call(
        body,
        grid=(NP // NB,),
        in_specs=[pl.BlockSpec((3, 2, PRS, 128), lambda i: (0, 0, i, 0)),
                  pl.BlockSpec((NB, 128), lambda i: (i, 0))],
        out_specs=pl.BlockSpec((NB, 128), lambda i: (i, 0)),
        out_shape=jax.ShapeDtypeStruct((NP, 128), jnp.float32),
    )(p, state)


def _mlp_call(state, W_l1, b_l1, W_l2, b_l2):
    def body(st_ref, w1_ref, b1_ref, w2_ref, b2_ref, o_ref):
        h = jnp.maximum(st_ref[:, 0:32] @ w1_ref[...] + b1_ref[...], 0.0)
        o_ref[...] = h @ w2_ref[...] + b2_ref[...]

    return pl.pallas_call(
        body,
        grid=(NP // NB,),
        in_specs=[pl.BlockSpec((NB, 128), lambda i: (i, 0)),
                  pl.BlockSpec((32, 16), lambda i: (0, 0)),
                  pl.BlockSpec((1, 16), lambda i: (0, 0)),
                  pl.BlockSpec((16, 8), lambda i: (0, 0)),
                  pl.BlockSpec((1, 8), lambda i: (0, 0))],
        out_specs=pl.BlockSpec((NB, LATENT), lambda i: (i, 0)),
        out_shape=jax.ShapeDtypeStruct((NP, LATENT), jnp.float32),
    )(state, W_l1, b_l1, W_l2, b_l2)


def _layer_weights(Wh, Ws, bs, Wv, Wg, bg, e_table, layer0):
    Ws_s = Ws[0:32]
    Tb = e_table @ Ws[32:40]
    Ws_rbf = Ws[40:50]
    Ws_vn = Ws[50:55]
    if layer0:
        Wmat = jnp.kron(Wh[4:5, :], jnp.eye(3, dtype=jnp.float32))
    else:
        Wmat = jnp.kron(Wh, jnp.eye(3, dtype=jnp.float32))
    Wv15 = jnp.kron(Wv, jnp.eye(3, dtype=jnp.float32))
    E43 = jnp.kron(jnp.eye(4, dtype=jnp.float32), jnp.ones((1, 3), jnp.float32))
    return (Ws_s, Tb, Ws_rbf, Ws_vn, bs[None, :], Wmat, Wv15, Wg, bg[None, :], E43)


def kernel(atom_types, atom_charges, bond_orders, coords, edge_index, a_table, c_table, e_table, W_ns, b_ns, Wh_0, Ws_0, bs_0, Wv_0, Wg_0, bg_0, Wh_1, Ws_1, bs_1, Wv_1, Wg_1, bg_1, Wh_2, Ws_2, bs_2, Wv_2, Wg_2, bg_2, W_l1, b_l1, W_l2, b_l2):
    pad = EP - E
    pad_idx = (jnp.arange(pad, dtype=jnp.int32) * 61) % N
    src_r = jnp.concatenate([edge_index[0].astype(jnp.int32), pad_idx]).reshape(ROWS, 128)
    dst_r = jnp.concatenate([edge_index[1].astype(jnp.int32), pad_idx]).reshape(ROWS, 128)
    bond_r = jnp.concatenate([bond_orders.astype(jnp.float32),
                              jnp.zeros((pad,), jnp.float32)]).reshape(ROWS, 128)
    feat = jnp.concatenate(
        [atom_types.astype(jnp.float32)[:, None],
         atom_charges.astype(jnp.float32)[:, None], coords,
         jnp.zeros((N, 3), jnp.float32)], axis=1)
    feat = jnp.concatenate([feat, jnp.zeros((NP - N, 8), jnp.float32)], axis=0)
    A2 = a_table @ W_ns[0:16]
    C2 = c_table @ W_ns[16:24]
    w0 = _layer_weights(Wh_0, Ws_0, bs_0, Wv_0, Wg_0, bg_0, e_table, False)
    w1 = _layer_weights(Wh_1, Ws_1, bs_1, Wv_1, Wg_1, bg_1, e_table, False)
    w2 = _layer_weights(Wh_2, Ws_2, bs_2, Wv_2, Wg_2, bg_2, e_table, False)

    state0 = _embed_call(feat, A2, C2, b_ns[None, :])

    ctab = state0[:, 44:60]
    gs0 = _sc_gather(ctab, src_r, 16)
    gd0 = _sc_gather(ctab, dst_r, 16)
    msg_init = _geom_call(gs0, gd0, bond_r)

    wstack = jax.tree.map(lambda *xs: jnp.stack(xs), w0, w1, w2)

    def layer(carry, wl):
        state, msg_prev = carry
        g = _sc_gather(state[:, 0:64], src_r, 64)
        msg = _edgeN_call(g, msg_prev, wl)
        p = _sc_scatter(msg, dst_r, 3)
        return (_update_call(p, state), msg), None

    (state3, _), _ = lax.scan(layer, (state0, msg_init), wstack)

    lat = _mlp_call(state3, W_l1, b_l1[None, :], W_l2, b_l2[None, :])

    atom_latents = lat[:N]
    mask = jnp.zeros((N,), dtype=bool)
    return (atom_latents, mask)


# speedup vs baseline: 33.0670x; 1.0052x over previous
import functools

import jax
import jax.numpy as jnp
from jax import lax
from jax.experimental import pallas as pl
from jax.experimental.pallas import tpu as pltpu
from jax.experimental.pallas import tpu_sc as plsc

N = 50000
E = 800000
SCALAR = 32
RBF_DIM = 10
RBF_DMAX = 32.0
LATENT = 8

EP = 819200
ROWS = EP // 128
NC = 2
NS = 16
NW = NC * NS
RPW = ROWS // NW
GK = 2
GSTEPS = RPW // GK
SK = 8
SSTEPS = RPW // SK

NP = 51200
NPS = NP // NS
PRS = NPS // 8
PROWS = NP // 8
ZCH = 640
NCH = NPS // ZCH
PCH = ZCH // 8
NB = NPS
RB = 16
EB = RB * 128

_sigma = RBF_DMAX / RBF_DIM


def _mesh():
    return plsc.VectorSubcoreMesh(core_axis_name="c", subcore_axis_name="s",
                                  num_cores=NC, num_subcores=NS)


def _sc_gather(table, idx_r, W):
    @functools.partial(
        pl.kernel,
        out_type=jax.ShapeDtypeStruct((ROWS, 128, 128), jnp.float32),
        mesh=_mesh(),
        compiler_params=pltpu.CompilerParams(use_tc_tiling_on_sc=False),
        scratch_types=[pltpu.VMEM((2, GK, 128), jnp.int32),
                       pltpu.VMEM((2, GK, 128, W), jnp.float32),
                       pltpu.SemaphoreType.DMA,
                       pltpu.SemaphoreType.DMA,
                       pltpu.SemaphoreType.DMA,
                       pltpu.SemaphoreType.DMA],
    )
    def k(tbl_h, idx_h, o_g, idxb, gbuf, gs0, gs1, os0, os1):
        wid = lax.axis_index("s") * NC + lax.axis_index("c")
        gsems = (gs0, gs1)
        osems = (os0, os1)

        def fire(s, b):
            rb = wid * RPW + s * GK
            pltpu.sync_copy(idx_h.at[pl.ds(rb, GK)], idxb.at[b])
            for j in range(GK):
                pltpu.async_copy(tbl_h.at[idxb.at[b, j]], gbuf.at[b, j],
                                 gsems[b])

        def drain_g(b):
            for j in range(GK):
                pltpu.make_async_copy(tbl_h.at[idxb.at[b, j]], gbuf.at[b, j],
                                      gsems[b]).wait()

        def flush(s, b):
            rb = wid * RPW + s * GK
            pltpu.async_copy(gbuf.at[b], o_g.at[pl.ds(rb, GK), :, pl.ds(0, W)],
                             osems[b])

        def drain_o(s, b):
            rb = wid * RPW + s * GK
            pltpu.make_async_copy(gbuf.at[b],
                                  o_g.at[pl.ds(rb, GK), :, pl.ds(0, W)],
                                  osems[b]).wait()

        fire(0, 0)

        def body(it, carry):
            for ph in range(2):
                s = it * 2 + ph
                b = ph
                nb = 1 - ph

                @pl.when(s >= 1)
                def _():
                    drain_o(s - 1, nb)

                @pl.when(s + 1 < GSTEPS)
                def _():
                    fire(s + 1, nb)

                drain_g(b)
                flush(s, b)
            return carry

        lax.fori_loop(0, GSTEPS // 2, body, 0)
        drain_o(GSTEPS - 1, (GSTEPS - 1) % 2)

    return k(table, idx_r)


def _sc_scatter(vals, dst_r, nphase):
    @functools.partial(
        pl.kernel,
        out_type=jax.ShapeDtypeStruct((nphase, NC, PROWS, 128), jnp.float32),
        mesh=_mesh(),
        compiler_params=pltpu.CompilerParams(use_tc_tiling_on_sc=False),
        scratch_types=[pltpu.VMEM((2, SK, 128), jnp.int32),
                       pltpu.VMEM((2, SK, 128, 16), jnp.float32),
                       pltpu.VMEM((ZCH, 16), jnp.float32),
                       pltpu.VMEM((ZCH, 16), jnp.float32),
                       pltpu.VMEM((PCH, 128), jnp.float32),
                       pltpu.SemaphoreType.DMA,
                       pltpu.SemaphoreType.DMA,
                       pltpu.VMEM_SHARED((NP, 16), jnp.float32)],
    )
    def k(vals_h, dst_h, out, idxb, vbuf, vz, vtmp, vstage, ls0, ls1, acc):
        cid = lax.axis_index("c")
        sid = lax.axis_index("s")
        lsems = (ls0, ls1)

        def zb(i, carry):
            vz[i, :] = jnp.zeros((16,), jnp.float32)
            return carry

        lax.fori_loop(0, ZCH, zb, 0)

        for p in range(nphase):
            def zacc(i, carry):
                pltpu.sync_copy(vz, acc.at[pl.ds(sid * NPS + i * ZCH, ZCH)])
                return carry

            lax.fori_loop(0, NCH, zacc, 0)
            plsc.subcore_barrier()

            def fire(s, b):
                rb = cid * (ROWS // NC) + sid * RPW + s * SK
                pltpu.sync_copy(dst_h.at[pl.ds(rb, SK)], idxb.at[b])
                pltpu.async_copy(
                    vals_h.at[pl.ds(rb, SK), :, pl.ds(p * 16, 16)],
                    vbuf.at[b], lsems[b])

            def drain(s, b):
                rb = cid * (ROWS // NC) + sid * RPW + s * SK
                pltpu.make_async_copy(
                    vals_h.at[pl.ds(rb, SK), :, pl.ds(p * 16, 16)],
                    vbuf.at[b], lsems[b]).wait()

            fire(0, 0)

            def body(it, carry):
                for ph in range(2):
                    s = it * 2 + ph
                    b = ph

                    @pl.when(s < SSTEPS)
                    def _():
                        @pl.when(s + 1 < SSTEPS)
                        def _():
                            fire(s + 1, 1 - ph)

                        drain(s, b)
                        for j in range(SK):
                            pltpu.sync_copy(vbuf.at[b, j],
                                            acc.at[idxb.at[b, j]], add=True)
                return carry

            lax.fori_loop(0, (SSTEPS + 1) // 2, body, 0)
            plsc.subcore_barrier()

            for ch in range(NCH):
                pltpu.sync_copy(acc.at[pl.ds(sid * NPS + ch * ZCH, ZCH)], vtmp)
                for g in range(8):
                    def rp(r, carry):
                        vstage[r, pl.ds(g * 16, 16)] = vtmp[g * PCH + r, :]
                        return carry

                    lax.fori_loop(0, PCH, rp, 0)
                pltpu.sync_copy(
                    vstage, out.at[p, cid, pl.ds(sid * PRS + ch * PCH, PCH)])

    return _call_scatter(k, vals, dst_r)


def _call_scatter(k, vals, dst_r):
    return k(vals, dst_r)


def _unpack_partials(pp):
    parts = []
    for ch in range(NCH):
        for g in range(8):
            r0, r1 = ch * PCH, (ch + 1) * PCH
            c0, c1 = g * 16, (g + 1) * 16
            parts.append(pp[0, r0:r1, c0:c1] + pp[1, r0:r1, c0:c1])
    return jnp.concatenate(parts, axis=0)


def _embed_call(feat, A2, C2, b_ns):
    def body(f_ref, a_ref, c_ref, b_ref, o_ref):
        t = f_ref[:, 0:1]
        q = f_ref[:, 1:2]
        oha = (t == lax.broadcasted_iota(jnp.int32, (NB, 10), 1).astype(
            jnp.float32)).astype(jnp.float32)
        ohc = (q == lax.broadcasted_iota(jnp.int32, (NB, 6), 1).astype(
            jnp.float32)).astype(jnp.float32)
        s0 = jnp.maximum(oha @ a_ref[...] + ohc @ c_ref[...] + b_ref[...], 0.0)
        o_ref[...] = jnp.concatenate(
            [s0, jnp.zeros((NB, 12), jnp.float32), f_ref[:, 2:5],
             jnp.zeros((NB, 81), jnp.float32)], axis=1)

    return pl.pallas_call(
        body,
        grid=(NP // NB,),
        in_specs=[pl.BlockSpec((NB, 8), lambda i: (i, 0)),
                  pl.BlockSpec((10, SCALAR), lambda i: (0, 0)),
                  pl.BlockSpec((6, SCALAR), lambda i: (0, 0)),
                  pl.BlockSpec((1, SCALAR), lambda i: (0, 0))],
        out_specs=pl.BlockSpec((NB, 128), lambda i: (i, 0)),
        out_shape=jax.ShapeDtypeStruct((NP, 128), jnp.float32),
    )(feat, A2, C2, b_ns)


def _edge_core(ss, v15, d, bond, pid, wts, with_mv):
    (Ws_s, Tb, Ws_rbf, Ws_vn, bs, Wmat, Wv15, Wg, bg, E43) = wts
    vh = v15
    vh2 = vh * vh
    vn = jnp.sqrt(vh2[:, 0:5] + vh2[:, 5:10] + vh2[:, 10:15] + 1e-8)
    mu = lax.broadcasted_iota(jnp.int32, (EB, RBF_DIM), 1).astype(
        jnp.float32) * (RBF_DMAX / (RBF_DIM - 1))
    rbf = jnp.exp(-(((d - mu) / _sigma) ** 2))
    oh = (bond == lax.broadcasted_iota(jnp.int32, (EB, 5), 1).astype(
        jnp.float32)).astype(jnp.float32)
    pre = ss @ Ws_s + oh @ Tb + rbf @ Ws_rbf + vn @ Ws_vn + bs
    ms = jnp.maximum(pre, 0.0)
    row = pid * EB + lax.broadcasted_iota(jnp.int32, (EB, 1), 0)
    live = (row < E).astype(jnp.float32)
    ms = ms * live
    if not with_mv:
        return ms, None
    gate = jax.nn.sigmoid(ms @ Wg + bg)
    mv = (vh @ Wv15) * (gate @ E43)
    mvc = jnp.concatenate([mv * live, live, jnp.zeros((EB, 3), jnp.float32)],
                          axis=1)
    return ms, mvc


def _msg_block(ms, mvc):
    if mvc is None:
        mvc = jnp.zeros((EB, 16), jnp.float32)
    return jnp.concatenate([ms, mvc, jnp.zeros((EB, 80), jnp.float32)],
                           axis=1).reshape(RB, 128, 128)


def _geom_call(gs, gd, bond_r):
    def body(gs_ref, gd_ref, bd_ref, msg_ref):
        gs2 = gs_ref[...].reshape(EB, 128)
        gd2 = gd_ref[...].reshape(EB, 128)
        dif = gs2[:, 0:3] - gd2[:, 0:3]
        d = jnp.sqrt(jnp.sum(dif * dif, axis=1, keepdims=True) + 1e-8)
        x_diff = dif / d
        geo = jnp.concatenate([x_diff, d], axis=1).reshape(RB, 128, 4)
        msg_ref[:, :, 48:52] = geo
        msg_ref[:, :, 52:53] = bd_ref[...][:, :, None]
        msg_ref[:, :, 53:56] = jnp.zeros((RB, 128, 3), jnp.float32)

    return pl.pallas_call(
        body,
        grid=(ROWS // RB,),
        in_specs=[pl.BlockSpec((RB, 128, 128), lambda i: (i, 0, 0)),
                  pl.BlockSpec((RB, 128, 128), lambda i: (i, 0, 0)),
                  pl.BlockSpec((RB, 128), lambda i: (i, 0))],
        out_specs=pl.BlockSpec((RB, 128, 128), lambda i: (i, 0, 0)),
        out_shape=jax.ShapeDtypeStruct((ROWS, 128, 128), jnp.float32),
    )(gs, gd, bond_r)


def _edgeN_call(g, msg_prev, wts):
    (Ws_s, Tb, Ws_rbf, Ws_vn, bs, Wh15, Wv15, Wg, bg, E43) = wts

    def body(g_ref, m_ref, ws_ref, tb_ref, wr_ref, wv_ref, bs_ref, wh_ref,
             wv15_ref, wg_ref, bg_ref, e43_ref, msg_ref):
        g2 = g_ref[...].reshape(EB, 128)
        m2 = m_ref[...].reshape(EB, 128)
        x_diff = m2[:, 48:51]
        d = m2[:, 51:52]
        bond = m2[:, 52:53]
        wh = wh_ref[...]
        vh = g2[:, 32:44] @ wh[0:12] + x_diff @ wh[12:15]
        w = (ws_ref[...], tb_ref[...], wr_ref[...], wv_ref[...],
             bs_ref[...], None, wv15_ref[...], wg_ref[...], bg_ref[...],
             e43_ref[...])
        ms, mvc = _edge_core(g2[:, 0:32], vh, d, bond, pl.program_id(0), w,
                             True)
        msg_ref[:, :, 0:48] = jnp.concatenate([ms, mvc],
                                              axis=1).reshape(RB, 128, 48)
        msg_ref[:, :, 48:56] = m_ref[:, :, 48:56]

    wspecs = [pl.BlockSpec((32, 32), lambda i: (0, 0)),
              pl.BlockSpec((5, 32), lambda i: (0, 0)),
              pl.BlockSpec((10, 32), lambda i: (0, 0)),
              pl.BlockSpec((5, 32), lambda i: (0, 0)),
              pl.BlockSpec((1, 32), lambda i: (0, 0)),
              pl.BlockSpec((15, 15), lambda i: (0, 0)),
              pl.BlockSpec((15, 12), lambda i: (0, 0)),
              pl.BlockSpec((32, 4), lambda i: (0, 0)),
              pl.BlockSpec((1, 4), lambda i: (0, 0)),
              pl.BlockSpec((4, 12), lambda i: (0, 0))]

    return pl.pallas_call(
        body,
        grid=(ROWS // RB,),
        in_specs=[pl.BlockSpec((RB, 128, 128), lambda i: (i, 0, 0)),
                  pl.BlockSpec((RB, 128, 128), lambda i: (i, 0, 0))] + wspecs,
        out_specs=pl.BlockSpec((RB, 128, 128), lambda i: (i, 0, 0)),
        out_shape=jax.ShapeDtypeStruct((ROWS, 128, 128), jnp.float32),
    )(g, msg_prev, Ws_s, Tb, Ws_rbf, Ws_vn, bs, Wh15, Wv15, Wg, bg, E43)


def _update_call(p, state):
    def body(p_ref, st_ref, o_ref):
        aggs = jnp.concatenate(
            [_unpack_partials(p_ref[0]), _unpack_partials(p_ref[1])], axis=1)
        aggv = _unpack_partials(p_ref[2])
        den = jnp.maximum(aggv[:, 12:13], 1.0)
        s1 = st_ref[:, 0:32] + aggs / den
        v1 = st_ref[:, 32:44] + aggv[:, 0:12] / den
        o_ref[...] = jnp.concatenate(
            [s1, v1, st_ref[:, 44:47], jnp.zeros((NB, 81), jnp.float32)],
            axis=1)

    return pl.pallas_call(
        body,
        grid=(NP // NB,),
        in_specs=[pl.BlockSpec((3, 2, PRS, 128), lambda i: (0, 0, i, 0)),
                  pl.BlockSpec((NB, 128), lambda i: (i, 0))],
        out_specs=pl.BlockSpec((NB, 128), lambda i: (i, 0)),
        out_shape=jax.ShapeDtypeStruct((NP, 128), jnp.float32),
    )(p, state)


def _mlp_call(state, W_l1, b_l1, W_l2, b_l2):
    def body(st_ref, w1_ref, b1_ref, w2_ref, b2_ref, o_ref):
        h = jnp.maximum(st_ref[:, 0:32] @ w1_ref[...] + b1_ref[...], 0.0)
        o_ref[...] = h @ w2_ref[...] + b2_ref[...]

    return pl.pallas_call(
        body,
        grid=(NP // NB,),
        in_specs=[pl.BlockSpec((NB, 128), lambda i: (i, 0)),
                  pl.BlockSpec((32, 16), lambda i: (0, 0)),
                  pl.BlockSpec((1, 16), lambda i: (0, 0)),
                  pl.BlockSpec((16, 8), lambda i: (0, 0)),
                  pl.BlockSpec((1, 8), lambda i: (0, 0))],
        out_specs=pl.BlockSpec((NB, LATENT), lambda i: (i, 0)),
        out_shape=jax.ShapeDtypeStruct((NP, LATENT), jnp.float32),
    )(state, W_l1, b_l1, W_l2, b_l2)


def _layer_weights(Wh, Ws, bs, Wv, Wg, bg, e_table, layer0):
    Ws_s = Ws[0:32]
    Tb = e_table @ Ws[32:40]
    Ws_rbf = Ws[40:50]
    Ws_vn = Ws[50:55]
    perm = jnp.array([3 * h + c for c in range(3) for h in range(5)],
                     dtype=jnp.int32)
    if layer0:
        Wmat = jnp.kron(Wh[4:5, :], jnp.eye(3, dtype=jnp.float32))
    else:
        Wmat = jnp.kron(Wh, jnp.eye(3, dtype=jnp.float32))
    Wmat = Wmat[:, perm]
    Wv15 = jnp.kron(Wv, jnp.eye(3, dtype=jnp.float32))[perm, :]
    E43 = jnp.kron(jnp.eye(4, dtype=jnp.float32), jnp.ones((1, 3), jnp.float32))
    return (Ws_s, Tb, Ws_rbf, Ws_vn, bs[None, :], Wmat, Wv15, Wg, bg[None, :], E43)


def kernel(atom_types, atom_charges, bond_orders, coords, edge_index, a_table, c_table, e_table, W_ns, b_ns, Wh_0, Ws_0, bs_0, Wv_0, Wg_0, bg_0, Wh_1, Ws_1, bs_1, Wv_1, Wg_1, bg_1, Wh_2, Ws_2, bs_2, Wv_2, Wg_2, bg_2, W_l1, b_l1, W_l2, b_l2):
    pad = EP - E
    pad_idx = (jnp.arange(pad, dtype=jnp.int32) * 61) % N
    src_r = jnp.concatenate([edge_index[0].astype(jnp.int32), pad_idx]).reshape(ROWS, 128)
    dst_r = jnp.concatenate([edge_index[1].astype(jnp.int32), pad_idx]).reshape(ROWS, 128)
    bond_r = jnp.concatenate([bond_orders.astype(jnp.float32),
                              jnp.zeros((pad,), jnp.float32)]).reshape(ROWS, 128)
    feat = jnp.concatenate(
        [atom_types.astype(jnp.float32)[:, None],
         atom_charges.astype(jnp.float32)[:, None], coords,
         jnp.zeros((N, 3), jnp.float32)], axis=1)
    feat = jnp.concatenate([feat, jnp.zeros((NP - N, 8), jnp.float32)], axis=0)
    A2 = a_table @ W_ns[0:16]
    C2 = c_table @ W_ns[16:24]
    w0 = _layer_weights(Wh_0, Ws_0, bs_0, Wv_0, Wg_0, bg_0, e_table, False)
    w1 = _layer_weights(Wh_1, Ws_1, bs_1, Wv_1, Wg_1, bg_1, e_table, False)
    w2 = _layer_weights(Wh_2, Ws_2, bs_2, Wv_2, Wg_2, bg_2, e_table, False)

    state0 = _embed_call(feat, A2, C2, b_ns[None, :])

    ctab = state0[:, 44:60]
    gs0 = _sc_gather(ctab, src_r, 16)
    gd0 = _sc_gather(ctab, dst_r, 16)
    msg_init = _geom_call(gs0, gd0, bond_r)

    wstack = jax.tree.map(lambda *xs: jnp.stack(xs), w0, w1, w2)

    def layer(carry, wl):
        state, msg_prev = carry
        g = _sc_gather(state[:, 0:64], src_r, 64)
        msg = _edgeN_call(g, msg_prev, wl)
        p = _sc_scatter(msg, dst_r, 3)
        return (_update_call(p, state), msg), None

    (state3, _), _ = lax.scan(layer, (state0, msg_init), wstack)

    lat = _mlp_call(state3, W_l1, b_l1[None, :], W_l2, b_l2[None, :])

    atom_latents = lat[:N]
    mask = jnp.zeros((N,), dtype=bool)
    return (atom_latents, mask)


# speedup vs baseline: 33.8848x; 1.0247x over previous
import functools

import jax
import jax.numpy as jnp
from jax import lax
from jax.experimental import pallas as pl
from jax.experimental.pallas import tpu as pltpu
from jax.experimental.pallas import tpu_sc as plsc

N = 50000
E = 800000
SCALAR = 32
RBF_DIM = 10
RBF_DMAX = 32.0
LATENT = 8

EP = 819200
ROWS = EP // 128
NC = 2
NS = 16
NW = NC * NS
RPW = ROWS // NW
GK = 4
GSTEPS = RPW // GK
SK = 8
SSTEPS = RPW // SK

NP = 51200
NPS = NP // NS
PRS = NPS // 8
PROWS = NP // 8
ZCH = 640
NCH = NPS // ZCH
PCH = ZCH // 8
NB = NPS
RB = 32
EB = RB * 128

_sigma = RBF_DMAX / RBF_DIM


def _mesh():
    return plsc.VectorSubcoreMesh(core_axis_name="c", subcore_axis_name="s",
                                  num_cores=NC, num_subcores=NS)


def _sc_gather(table, idx_r, W):
    @functools.partial(
        pl.kernel,
        out_type=jax.ShapeDtypeStruct((ROWS, 128, 128), jnp.float32),
        mesh=_mesh(),
        compiler_params=pltpu.CompilerParams(use_tc_tiling_on_sc=False),
        scratch_types=[pltpu.VMEM((2, GK, 128), jnp.int32),
                       pltpu.VMEM((2, GK, 128, W), jnp.float32),
                       pltpu.SemaphoreType.DMA,
                       pltpu.SemaphoreType.DMA,
                       pltpu.SemaphoreType.DMA,
                       pltpu.SemaphoreType.DMA],
    )
    def k(tbl_h, idx_h, o_g, idxb, gbuf, gs0, gs1, os0, os1):
        wid = lax.axis_index("s") * NC + lax.axis_index("c")
        gsems = (gs0, gs1)
        osems = (os0, os1)

        def fire(s, b):
            rb = wid * RPW + s * GK
            pltpu.sync_copy(idx_h.at[pl.ds(rb, GK)], idxb.at[b])
            for j in range(GK):
                pltpu.async_copy(tbl_h.at[idxb.at[b, j]], gbuf.at[b, j],
                                 gsems[b])

        def drain_g(b):
            for j in range(GK):
                pltpu.make_async_copy(tbl_h.at[idxb.at[b, j]], gbuf.at[b, j],
                                      gsems[b]).wait()

        def flush(s, b):
            rb = wid * RPW + s * GK
            pltpu.async_copy(gbuf.at[b], o_g.at[pl.ds(rb, GK), :, pl.ds(0, W)],
                             osems[b])

        def drain_o(s, b):
            rb = wid * RPW + s * GK
            pltpu.make_async_copy(gbuf.at[b],
                                  o_g.at[pl.ds(rb, GK), :, pl.ds(0, W)],
                                  osems[b]).wait()

        fire(0, 0)

        def body(it, carry):
            for ph in range(2):
                s = it * 2 + ph
                b = ph
                nb = 1 - ph

                @pl.when(s >= 1)
                def _():
                    drain_o(s - 1, nb)

                @pl.when(s + 1 < GSTEPS)
                def _():
                    fire(s + 1, nb)

                drain_g(b)
                flush(s, b)
            return carry

        lax.fori_loop(0, GSTEPS // 2, body, 0)
        drain_o(GSTEPS - 1, (GSTEPS - 1) % 2)

    return k(table, idx_r)


def _sc_scatter(vals, dst_r, nphase):
    @functools.partial(
        pl.kernel,
        out_type=jax.ShapeDtypeStruct((nphase, NC, PROWS, 128), jnp.float32),
        mesh=_mesh(),
        compiler_params=pltpu.CompilerParams(use_tc_tiling_on_sc=False),
        scratch_types=[pltpu.VMEM((2, SK, 128), jnp.int32),
                       pltpu.VMEM((2, SK, 128, 16), jnp.float32),
                       pltpu.VMEM((ZCH, 16), jnp.float32),
                       pltpu.VMEM((ZCH, 16), jnp.float32),
                       pltpu.VMEM((PCH, 128), jnp.float32),
                       pltpu.SemaphoreType.DMA,
                       pltpu.SemaphoreType.DMA,
                       pltpu.VMEM_SHARED((NP, 16), jnp.float32)],
    )
    def k(vals_h, dst_h, out, idxb, vbuf, vz, vtmp, vstage, ls0, ls1, acc):
        cid = lax.axis_index("c")
        sid = lax.axis_index("s")
        lsems = (ls0, ls1)

        def zb(i, carry):
            vz[i, :] = jnp.zeros((16,), jnp.float32)
            return carry

        lax.fori_loop(0, ZCH, zb, 0)

        for p in range(nphase):
            def zacc(i, carry):
                pltpu.sync_copy(vz, acc.at[pl.ds(sid * NPS + i * ZCH, ZCH)])
                return carry

            lax.fori_loop(0, NCH, zacc, 0)
            plsc.subcore_barrier()

            def fire(s, b):
                rb = cid * (ROWS // NC) + sid * RPW + s * SK
                pltpu.sync_copy(dst_h.at[pl.ds(rb, SK)], idxb.at[b])
                pltpu.async_copy(
                    vals_h.at[pl.ds(rb, SK), :, pl.ds(p * 16, 16)],
                    vbuf.at[b], lsems[b])

            def drain(s, b):
                rb = cid * (ROWS // NC) + sid * RPW + s * SK
                pltpu.make_async_copy(
                    vals_h.at[pl.ds(rb, SK), :, pl.ds(p * 16, 16)],
                    vbuf.at[b], lsems[b]).wait()

            fire(0, 0)

            def body(it, carry):
                for ph in range(2):
                    s = it * 2 + ph
                    b = ph

                    @pl.when(s < SSTEPS)
                    def _():
                        @pl.when(s + 1 < SSTEPS)
                        def _():
                            fire(s + 1, 1 - ph)

                        drain(s, b)
                        for j in range(SK):
                            pltpu.sync_copy(vbuf.at[b, j],
                                            acc.at[idxb.at[b, j]], add=True)
                return carry

            lax.fori_loop(0, (SSTEPS + 1) // 2, body, 0)
            plsc.subcore_barrier()

            for ch in range(NCH):
                pltpu.sync_copy(acc.at[pl.ds(sid * NPS + ch * ZCH, ZCH)], vtmp)
                for g in range(8):
                    def rp(r, carry):
                        vstage[r, pl.ds(g * 16, 16)] = vtmp[g * PCH + r, :]
                        return carry

                    lax.fori_loop(0, PCH, rp, 0)
                pltpu.sync_copy(
                    vstage, out.at[p, cid, pl.ds(sid * PRS + ch * PCH, PCH)])

    return _call_scatter(k, vals, dst_r)


def _call_scatter(k, vals, dst_r):
    return k(vals, dst_r)


def _unpack_partials(pp):
    parts = []
    for ch in range(NCH):
        for g in range(8):
            r0, r1 = ch * PCH, (ch + 1) * PCH
            c0, c1 = g * 16, (g + 1) * 16
            parts.append(pp[0, r0:r1, c0:c1] + pp[1, r0:r1, c0:c1])
    return jnp.concatenate(parts, axis=0)


def _embed_call(feat, A2, C2, b_ns):
    def body(f_ref, a_ref, c_ref, b_ref, o_ref):
        t = f_ref[:, 0:1]
        q = f_ref[:, 1:2]
        oha = (t == lax.broadcasted_iota(jnp.int32, (NB, 10), 1).astype(
            jnp.float32)).astype(jnp.float32)
        ohc = (q == lax.broadcasted_iota(jnp.int32, (NB, 6), 1).astype(
            jnp.float32)).astype(jnp.float32)
        s0 = jnp.maximum(oha @ a_ref[...] + ohc @ c_ref[...] + b_ref[...], 0.0)
        o_ref[...] = jnp.concatenate(
            [s0, jnp.zeros((NB, 12), jnp.float32), f_ref[:, 2:5],
             jnp.zeros((NB, 81), jnp.float32)], axis=1)

    return pl.pallas_call(
        body,
        grid=(NP // NB,),
        in_specs=[pl.BlockSpec((NB, 8), lambda i: (i, 0)),
                  pl.BlockSpec((10, SCALAR), lambda i: (0, 0)),
                  pl.BlockSpec((6, SCALAR), lambda i: (0, 0)),
                  pl.BlockSpec((1, SCALAR), lambda i: (0, 0))],
        out_specs=pl.BlockSpec((NB, 128), lambda i: (i, 0)),
        out_shape=jax.ShapeDtypeStruct((NP, 128), jnp.float32),
    )(feat, A2, C2, b_ns)


def _edge_core(ss, v15, d, bond, pid, wts, with_mv):
    (Ws_s, Tb, Ws_rbf, Ws_vn, bs, Wmat, Wv15, Wg, bg, E43) = wts
    vh = v15
    vh2 = vh * vh
    vn = jnp.sqrt(vh2[:, 0:5] + vh2[:, 5:10] + vh2[:, 10:15] + 1e-8)
    mu = lax.broadcasted_iota(jnp.int32, (EB, RBF_DIM), 1).astype(
        jnp.float32) * (RBF_DMAX / (RBF_DIM - 1))
    rbf = jnp.exp(-(((d - mu) / _sigma) ** 2))
    oh = (bond == lax.broadcasted_iota(jnp.int32, (EB, 5), 1).astype(
        jnp.float32)).astype(jnp.float32)
    pre = ss @ Ws_s + oh @ Tb + rbf @ Ws_rbf + vn @ Ws_vn + bs
    ms = jnp.maximum(pre, 0.0)
    row = pid * EB + lax.broadcasted_iota(jnp.int32, (EB, 1), 0)
    live = (row < E).astype(jnp.float32)
    ms = ms * live
    if not with_mv:
        return ms, None
    gate = jax.nn.sigmoid(ms @ Wg + bg)
    mv = (vh @ Wv15) * (gate @ E43)
    mvc = jnp.concatenate([mv * live, live, jnp.zeros((EB, 3), jnp.float32)],
                          axis=1)
    return ms, mvc


def _msg_block(ms, mvc):
    if mvc is None:
        mvc = jnp.zeros((EB, 16), jnp.float32)
    return jnp.concatenate([ms, mvc, jnp.zeros((EB, 80), jnp.float32)],
                           axis=1).reshape(RB, 128, 128)


def _geom_call(gs, gd, bond_r):
    def body(gs_ref, gd_ref, bd_ref, msg_ref):
        gs2 = gs_ref[...].reshape(EB, 128)
        gd2 = gd_ref[...].reshape(EB, 128)
        dif = gs2[:, 0:3] - gd2[:, 0:3]
        d = jnp.sqrt(jnp.sum(dif * dif, axis=1, keepdims=True) + 1e-8)
        x_diff = dif / d
        geo = jnp.concatenate([x_diff, d], axis=1).reshape(RB, 128, 4)
        msg_ref[:, :, 48:52] = geo
        msg_ref[:, :, 52:53] = bd_ref[...][:, :, None]
        msg_ref[:, :, 53:56] = jnp.zeros((RB, 128, 3), jnp.float32)

    return pl.pallas_call(
        body,
        grid=(ROWS // RB,),
        in_specs=[pl.BlockSpec((RB, 128, 128), lambda i: (i, 0, 0)),
                  pl.BlockSpec((RB, 128, 128), lambda i: (i, 0, 0)),
                  pl.BlockSpec((RB, 128), lambda i: (i, 0))],
        out_specs=pl.BlockSpec((RB, 128, 128), lambda i: (i, 0, 0)),
        out_shape=jax.ShapeDtypeStruct((ROWS, 128, 128), jnp.float32),
    )(gs, gd, bond_r)


def _edgeN_call(g, msg_prev, wts):
    (Ws_s, Tb, Ws_rbf, Ws_vn, bs, Wh15, Wv15, Wg, bg, E43) = wts

    def body(g_ref, m_ref, ws_ref, tb_ref, wr_ref, wv_ref, bs_ref, wh_ref,
             wv15_ref, wg_ref, bg_ref, e43_ref, msg_ref):
        g2 = g_ref[...].reshape(EB, 128)
        m2 = m_ref[...].reshape(EB, 128)
        x_diff = m2[:, 48:51]
        d = m2[:, 51:52]
        bond = m2[:, 52:53]
        wh = wh_ref[...]
        vh = g2[:, 32:44] @ wh[0:12] + x_diff @ wh[12:15]
        w = (ws_ref[...], tb_ref[...], wr_ref[...], wv_ref[...],
             bs_ref[...], None, wv15_ref[...], wg_ref[...], bg_ref[...],
             e43_ref[...])
        ms, mvc = _edge_core(g2[:, 0:32], vh, d, bond, pl.program_id(0), w,
                             True)
        msg_ref[:, :, 0:48] = jnp.concatenate([ms, mvc],
                                              axis=1).reshape(RB, 128, 48)
        msg_ref[:, :, 48:56] = m_ref[:, :, 48:56]

    wspecs = [pl.BlockSpec((32, 32), lambda i: (0, 0)),
              pl.BlockSpec((5, 32), lambda i: (0, 0)),
              pl.BlockSpec((10, 32), lambda i: (0, 0)),
              pl.BlockSpec((5, 32), lambda i: (0, 0)),
              pl.BlockSpec((1, 32), lambda i: (0, 0)),
              pl.BlockSpec((15, 15), lambda i: (0, 0)),
              pl.BlockSpec((15, 12), lambda i: (0, 0)),
              pl.BlockSpec((32, 4), lambda i: (0, 0)),
              pl.BlockSpec((1, 4), lambda i: (0, 0)),
              pl.BlockSpec((4, 12), lambda i: (0, 0))]

    return pl.pallas_call(
        body,
        grid=(ROWS // RB,),
        in_specs=[pl.BlockSpec((RB, 128, 128), lambda i: (i, 0, 0)),
                  pl.BlockSpec((RB, 128, 128), lambda i: (i, 0, 0))] + wspecs,
        out_specs=pl.BlockSpec((RB, 128, 128), lambda i: (i, 0, 0)),
        out_shape=jax.ShapeDtypeStruct((ROWS, 128, 128), jnp.float32),
    )(g, msg_prev, Ws_s, Tb, Ws_rbf, Ws_vn, bs, Wh15, Wv15, Wg, bg, E43)


def _update_call(p, state):
    def body(p_ref, st_ref, o_ref):
        aggs = jnp.concatenate(
            [_unpack_partials(p_ref[0]), _unpack_partials(p_ref[1])], axis=1)
        aggv = _unpack_partials(p_ref[2])
        den = jnp.maximum(aggv[:, 12:13], 1.0)
        s1 = st_ref[:, 0:32] + aggs / den
        v1 = st_ref[:, 32:44] + aggv[:, 0:12] / den
        o_ref[...] = jnp.concatenate(
            [s1, v1, st_ref[:, 44:47], jnp.zeros((NB, 81), jnp.float32)],
            axis=1)

    return pl.pallas_call(
        body,
        grid=(NP // NB,),
        in_specs=[pl.BlockSpec((3, 2, PRS, 128), lambda i: (0, 0, i, 0)),
                  pl.BlockSpec((NB, 128), lambda i: (i, 0))],
        out_specs=pl.BlockSpec((NB, 128), lambda i: (i, 0)),
        out_shape=jax.ShapeDtypeStruct((NP, 128), jnp.float32),
    )(p, state)


def _mlp_call(state, W_l1, b_l1, W_l2, b_l2):
    def body(st_ref, w1_ref, b1_ref, w2_ref, b2_ref, o_ref):
        h = jnp.maximum(st_ref[:, 0:32] @ w1_ref[...] + b1_ref[...], 0.0)
        o_ref[...] = h @ w2_ref[...] + b2_ref[...]

    return pl.pallas_call(
        body,
        grid=(NP // NB,),
        in_specs=[pl.BlockSpec((NB, 128), lambda i: (i, 0)),
                  pl.BlockSpec((32, 16), lambda i: (0, 0)),
                  pl.BlockSpec((1, 16), lambda i: (0, 0)),
                  pl.BlockSpec((16, 8), lambda i: (0, 0)),
                  pl.BlockSpec((1, 8), lambda i: (0, 0))],
        out_specs=pl.BlockSpec((NB, LATENT), lambda i: (i, 0)),
        out_shape=jax.ShapeDtypeStruct((NP, LATENT), jnp.float32),
    )(state, W_l1, b_l1, W_l2, b_l2)


def _layer_weights(Wh, Ws, bs, Wv, Wg, bg, e_table, layer0):
    Ws_s = Ws[0:32]
    Tb = e_table @ Ws[32:40]
    Ws_rbf = Ws[40:50]
    Ws_vn = Ws[50:55]
    perm = jnp.array([3 * h + c for c in range(3) for h in range(5)],
                     dtype=jnp.int32)
    if layer0:
        Wmat = jnp.kron(Wh[4:5, :], jnp.eye(3, dtype=jnp.float32))
    else:
        Wmat = jnp.kron(Wh, jnp.eye(3, dtype=jnp.float32))
    Wmat = Wmat[:, perm]
    Wv15 = jnp.kron(Wv, jnp.eye(3, dtype=jnp.float32))[perm, :]
    E43 = jnp.kron(jnp.eye(4, dtype=jnp.float32), jnp.ones((1, 3), jnp.float32))
    return (Ws_s, Tb, Ws_rbf, Ws_vn, bs[None, :], Wmat, Wv15, Wg, bg[None, :], E43)


def kernel(atom_types, atom_charges, bond_orders, coords, edge_index, a_table, c_table, e_table, W_ns, b_ns, Wh_0, Ws_0, bs_0, Wv_0, Wg_0, bg_0, Wh_1, Ws_1, bs_1, Wv_1, Wg_1, bg_1, Wh_2, Ws_2, bs_2, Wv_2, Wg_2, bg_2, W_l1, b_l1, W_l2, b_l2):
    pad = EP - E
    pad_idx = (jnp.arange(pad, dtype=jnp.int32) * 61) % N
    src_r = jnp.concatenate([edge_index[0].astype(jnp.int32), pad_idx]).reshape(ROWS, 128)
    dst_r = jnp.concatenate([edge_index[1].astype(jnp.int32), pad_idx]).reshape(ROWS, 128)
    bond_r = jnp.concatenate([bond_orders.astype(jnp.float32),
                              jnp.zeros((pad,), jnp.float32)]).reshape(ROWS, 128)
    feat = jnp.concatenate(
        [atom_types.astype(jnp.float32)[:, None],
         atom_charges.astype(jnp.float32)[:, None], coords,
         jnp.zeros((N, 3), jnp.float32)], axis=1)
    feat = jnp.concatenate([feat, jnp.zeros((NP - N, 8), jnp.float32)], axis=0)
    A2 = a_table @ W_ns[0:16]
    C2 = c_table @ W_ns[16:24]
    w0 = _layer_weights(Wh_0, Ws_0, bs_0, Wv_0, Wg_0, bg_0, e_table, False)
    w1 = _layer_weights(Wh_1, Ws_1, bs_1, Wv_1, Wg_1, bg_1, e_table, False)
    w2 = _layer_weights(Wh_2, Ws_2, bs_2, Wv_2, Wg_2, bg_2, e_table, False)

    state0 = _embed_call(feat, A2, C2, b_ns[None, :])

    ctab = state0[:, 44:60]
    gs0 = _sc_gather(ctab, src_r, 16)
    gd0 = _sc_gather(ctab, dst_r, 16)
    msg_init = _geom_call(gs0, gd0, bond_r)

    wstack = jax.tree.map(lambda *xs: jnp.stack(xs), w0, w1, w2)

    def layer(carry, wl):
        state, msg_prev = carry
        g = _sc_gather(state[:, 0:64], src_r, 64)
        msg = _edgeN_call(g, msg_prev, wl)
        p = _sc_scatter(msg, dst_r, 3)
        return (_update_call(p, state), msg), None

    (state3, _), _ = lax.scan(layer, (state0, msg_init), wstack)

    lat = _mlp_call(state3, W_l1, b_l1[None, :], W_l2, b_l2[None, :])

    atom_latents = lat[:N]
    mask = jnp.zeros((N,), dtype=bool)
    return (atom_latents, mask)


# speedup vs baseline: 39.3584x; 1.1615x over previous
import functools

import jax
import jax.numpy as jnp
from jax import lax
from jax.experimental import pallas as pl
from jax.experimental.pallas import tpu as pltpu
from jax.experimental.pallas import tpu_sc as plsc

N = 50000
E = 800000
SCALAR = 32
RBF_DIM = 10
RBF_DMAX = 32.0
LATENT = 8

EP = 819200
ROWS = EP // 128
NC = 2
NS = 16
NW = NC * NS
RPW = ROWS // NW
GK = 4
GSTEPS = RPW // GK
SK = 8
SSTEPS = RPW // SK

NP = 51200
NPS = NP // NS
PRS = NPS // 8
PROWS = NP // 8
ZCH = 640
NCH = NPS // ZCH
PCH = ZCH // 8
NB = NPS
RB = 32
EB = RB * 128

_sigma = RBF_DMAX / RBF_DIM


def _mesh():
    return plsc.VectorSubcoreMesh(core_axis_name="c", subcore_axis_name="s",
                                  num_cores=NC, num_subcores=NS)


def _sc_gather(table, idx_r, W):
    @functools.partial(
        pl.kernel,
        out_type=jax.ShapeDtypeStruct((ROWS, 128, 128), jnp.float32),
        mesh=_mesh(),
        compiler_params=pltpu.CompilerParams(use_tc_tiling_on_sc=False),
        scratch_types=[pltpu.VMEM((2, GK, 128), jnp.int32),
                       pltpu.VMEM((2, GK, 128, W), jnp.float32),
                       pltpu.SemaphoreType.DMA,
                       pltpu.SemaphoreType.DMA,
                       pltpu.SemaphoreType.DMA,
                       pltpu.SemaphoreType.DMA],
    )
    def k(tbl_h, idx_h, o_g, idxb, gbuf, gs0, gs1, os0, os1):
        wid = lax.axis_index("s") * NC + lax.axis_index("c")
        gsems = (gs0, gs1)
        osems = (os0, os1)

        def fire(s, b):
            rb = wid * RPW + s * GK
            pltpu.sync_copy(idx_h.at[pl.ds(rb, GK)], idxb.at[b])
            for j in range(GK):
                pltpu.async_copy(tbl_h.at[idxb.at[b, j]], gbuf.at[b, j],
                                 gsems[b])

        def drain_g(b):
            for j in range(GK):
                pltpu.make_async_copy(tbl_h.at[idxb.at[b, j]], gbuf.at[b, j],
                                      gsems[b]).wait()

        def flush(s, b):
            rb = wid * RPW + s * GK
            pltpu.async_copy(gbuf.at[b], o_g.at[pl.ds(rb, GK), :, pl.ds(0, W)],
                             osems[b])

        def drain_o(s, b):
            rb = wid * RPW + s * GK
            pltpu.make_async_copy(gbuf.at[b],
                                  o_g.at[pl.ds(rb, GK), :, pl.ds(0, W)],
                                  osems[b]).wait()

        fire(0, 0)

        def body(it, carry):
            for ph in range(2):
                s = it * 2 + ph
                b = ph
                nb = 1 - ph

                @pl.when(s >= 1)
                def _():
                    drain_o(s - 1, nb)

                @pl.when(s + 1 < GSTEPS)
                def _():
                    fire(s + 1, nb)

                drain_g(b)
                flush(s, b)
            return carry

        lax.fori_loop(0, GSTEPS // 2, body, 0)
        drain_o(GSTEPS - 1, (GSTEPS - 1) % 2)

    return k(table, idx_r)


def _sc_scatter(vals, dst_r, nphase):
    @functools.partial(
        pl.kernel,
        out_type=jax.ShapeDtypeStruct((nphase, NC, PROWS, 128), jnp.float32),
        mesh=_mesh(),
        compiler_params=pltpu.CompilerParams(use_tc_tiling_on_sc=False),
        scratch_types=[pltpu.VMEM((2, SK, 128), jnp.int32),
                       pltpu.VMEM((2, SK, 128, 16), jnp.float32),
                       pltpu.VMEM((ZCH, 16), jnp.float32),
                       pltpu.VMEM((ZCH, 16), jnp.float32),
                       pltpu.VMEM((PCH, 128), jnp.float32),
                       pltpu.SemaphoreType.DMA,
                       pltpu.SemaphoreType.DMA,
                       pltpu.VMEM_SHARED((NP, 16), jnp.float32)],
    )
    def k(vals_h, dst_h, out, idxb, vbuf, vz, vtmp, vstage, ls0, ls1, acc):
        cid = lax.axis_index("c")
        sid = lax.axis_index("s")
        lsems = (ls0, ls1)

        def zb(i, carry):
            vz[i, :] = jnp.zeros((16,), jnp.float32)
            return carry

        lax.fori_loop(0, ZCH, zb, 0)

        for p in range(nphase):
            def zacc(i, carry):
                pltpu.sync_copy(vz, acc.at[pl.ds(sid * NPS + i * ZCH, ZCH)])
                return carry

            lax.fori_loop(0, NCH, zacc, 0)
            plsc.subcore_barrier()

            def fire(s, b):
                rb = cid * (ROWS // NC) + sid * RPW + s * SK
                pltpu.sync_copy(dst_h.at[pl.ds(rb, SK)], idxb.at[b])
                pltpu.async_copy(
                    vals_h.at[pl.ds(rb, SK), :, pl.ds(p * 16, 16)],
                    vbuf.at[b], lsems[b])

            def drain(s, b):
                rb = cid * (ROWS // NC) + sid * RPW + s * SK
                pltpu.make_async_copy(
                    vals_h.at[pl.ds(rb, SK), :, pl.ds(p * 16, 16)],
                    vbuf.at[b], lsems[b]).wait()

            fire(0, 0)

            def body(it, carry):
                for ph in range(2):
                    s = it * 2 + ph
                    b = ph

                    @pl.when(s < SSTEPS)
                    def _():
                        @pl.when(s + 1 < SSTEPS)
                        def _():
                            fire(s + 1, 1 - ph)

                        drain(s, b)
                        for j in range(SK):
                            pltpu.sync_copy(vbuf.at[b, j],
                                            acc.at[idxb.at[b, j]], add=True)
                return carry

            lax.fori_loop(0, (SSTEPS + 1) // 2, body, 0)
            plsc.subcore_barrier()

            for ch in range(NCH):
                pltpu.sync_copy(acc.at[pl.ds(sid * NPS + ch * ZCH, ZCH)], vtmp)
                for g in range(8):
                    def rp(r, carry):
                        vstage[r, pl.ds(g * 16, 16)] = vtmp[g * PCH + r, :]
                        return carry

                    lax.fori_loop(0, PCH, rp, 0)
                pltpu.sync_copy(
                    vstage, out.at[p, cid, pl.ds(sid * PRS + ch * PCH, PCH)])

    return _call_scatter(k, vals, dst_r)


def _call_scatter(k, vals, dst_r):
    return k(vals, dst_r)


def _unpack_partials(pp):
    parts = []
    for ch in range(NCH):
        for g in range(8):
            r0, r1 = ch * PCH, (ch + 1) * PCH
            c0, c1 = g * 16, (g + 1) * 16
            parts.append(pp[0, r0:r1, c0:c1] + pp[1, r0:r1, c0:c1])
    return jnp.concatenate(parts, axis=0)


def _embed_call(feat, A2, C2, b_ns):
    def body(f_ref, a_ref, c_ref, b_ref, o_ref):
        t = f_ref[:, 0:1]
        q = f_ref[:, 1:2]
        oha = (t == lax.broadcasted_iota(jnp.int32, (NB, 10), 1).astype(
            jnp.float32)).astype(jnp.float32)
        ohc = (q == lax.broadcasted_iota(jnp.int32, (NB, 6), 1).astype(
            jnp.float32)).astype(jnp.float32)
        s0 = jnp.maximum(oha @ a_ref[...] + ohc @ c_ref[...] + b_ref[...], 0.0)
        o_ref[...] = jnp.concatenate(
            [s0, jnp.zeros((NB, 12), jnp.float32), f_ref[:, 2:5],
             jnp.zeros((NB, 81), jnp.float32)], axis=1)

    return pl.pallas_call(
        body,
        grid=(NP // NB,),
        in_specs=[pl.BlockSpec((NB, 8), lambda i: (i, 0)),
                  pl.BlockSpec((10, SCALAR), lambda i: (0, 0)),
                  pl.BlockSpec((6, SCALAR), lambda i: (0, 0)),
                  pl.BlockSpec((1, SCALAR), lambda i: (0, 0))],
        out_specs=pl.BlockSpec((NB, 128), lambda i: (i, 0)),
        out_shape=jax.ShapeDtypeStruct((NP, 128), jnp.float32),
    )(feat, A2, C2, b_ns)


def _edge_core(ss, v15, d, bond, pid, wts, with_mv):
    (Ws_s, Tb, Ws_rbf, Ws_vn, bs, Wmat, Wv15, Wg, bg, E43) = wts
    vh = v15
    vh2 = vh * vh
    vn = jnp.sqrt(vh2[:, 0:5] + vh2[:, 5:10] + vh2[:, 10:15] + 1e-8)
    mu = lax.broadcasted_iota(jnp.int32, (EB, RBF_DIM), 1).astype(
        jnp.float32) * (RBF_DMAX / (RBF_DIM - 1))
    rbf = jnp.exp(-(((d - mu) / _sigma) ** 2))
    oh = (bond == lax.broadcasted_iota(jnp.int32, (EB, 5), 1).astype(
        jnp.float32)).astype(jnp.float32)
    pre = ss @ Ws_s + oh @ Tb + rbf @ Ws_rbf + vn @ Ws_vn + bs
    ms = jnp.maximum(pre, 0.0)
    row = pid * EB + lax.broadcasted_iota(jnp.int32, (EB, 1), 0)
    live = (row < E).astype(jnp.float32)
    ms = ms * live
    if not with_mv:
        return ms, None
    gate = jax.nn.sigmoid(ms @ Wg + bg)
    mv = (vh @ Wv15) * (gate @ E43)
    mvc = jnp.concatenate([mv * live, live, jnp.zeros((EB, 3), jnp.float32)],
                          axis=1)
    return ms, mvc


def _msg_block(ms, mvc):
    if mvc is None:
        mvc = jnp.zeros((EB, 16), jnp.float32)
    return jnp.concatenate([ms, mvc, jnp.zeros((EB, 80), jnp.float32)],
                           axis=1).reshape(RB, 128, 128)


def _geom_call(gs, gd, bond_r):
    def body(gs_ref, gd_ref, bd_ref, msg_ref):
        gs2 = gs_ref[...].reshape(EB, 128)
        gd2 = gd_ref[...].reshape(EB, 128)
        dif = gs2[:, 0:3] - gd2[:, 0:3]
        d = jnp.sqrt(jnp.sum(dif * dif, axis=1, keepdims=True) + 1e-8)
        x_diff = dif / d
        geo = jnp.concatenate([x_diff, d], axis=1).reshape(RB, 128, 4)
        msg_ref[:, :, 48:52] = geo
        msg_ref[:, :, 52:53] = bd_ref[...][:, :, None]
        msg_ref[:, :, 53:56] = jnp.zeros((RB, 128, 3), jnp.float32)

    return pl.pallas_call(
        body,
        grid=(ROWS // RB,),
        in_specs=[pl.BlockSpec((RB, 128, 128), lambda i: (i, 0, 0)),
                  pl.BlockSpec((RB, 128, 128), lambda i: (i, 0, 0)),
                  pl.BlockSpec((RB, 128), lambda i: (i, 0))],
        out_specs=pl.BlockSpec((RB, 128, 128), lambda i: (i, 0, 0)),
        out_shape=jax.ShapeDtypeStruct((ROWS, 128, 128), jnp.float32),
    )(gs, gd, bond_r)


def _edgeN_call(g, msg_prev, wts):
    (Ws_s, Tb, Ws_rbf, Ws_vn, bs, Wh15, Wv15, Wg, bg, E43) = wts

    def body(g_ref, m_ref, ws_ref, tb_ref, wr_ref, wv_ref, bs_ref, wh_ref,
             wv15_ref, wg_ref, bg_ref, e43_ref, msg_ref):
        g2 = g_ref[...].reshape(EB, 128)
        m2 = m_ref[...].reshape(EB, 128)
        x_diff = m2[:, 48:51]
        d = m2[:, 51:52]
        bond = m2[:, 52:53]
        wh = wh_ref[...]
        vh = g2[:, 32:44] @ wh[0:12] + x_diff @ wh[12:15]
        w = (ws_ref[...], tb_ref[...], wr_ref[...], wv_ref[...],
             bs_ref[...], None, wv15_ref[...], wg_ref[...], bg_ref[...],
             e43_ref[...])
        ms, mvc = _edge_core(g2[:, 0:32], vh, d, bond, pl.program_id(0), w,
                             True)
        msg_ref[:, :, 0:48] = jnp.concatenate([ms, mvc],
                                              axis=1).reshape(RB, 128, 48)
        msg_ref[:, :, 48:56] = m_ref[:, :, 48:56]

    wspecs = [pl.BlockSpec((32, 32), lambda i: (0, 0)),
              pl.BlockSpec((5, 32), lambda i: (0, 0)),
              pl.BlockSpec((10, 32), lambda i: (0, 0)),
              pl.BlockSpec((5, 32), lambda i: (0, 0)),
              pl.BlockSpec((1, 32), lambda i: (0, 0)),
              pl.BlockSpec((15, 15), lambda i: (0, 0)),
              pl.BlockSpec((15, 12), lambda i: (0, 0)),
              pl.BlockSpec((32, 4), lambda i: (0, 0)),
              pl.BlockSpec((1, 4), lambda i: (0, 0)),
              pl.BlockSpec((4, 12), lambda i: (0, 0))]

    return pl.pallas_call(
        body,
        grid=(ROWS // RB,),
        in_specs=[pl.BlockSpec((RB, 128, 128), lambda i: (i, 0, 0)),
                  pl.BlockSpec((RB, 128, 128), lambda i: (i, 0, 0))] + wspecs,
        out_specs=pl.BlockSpec((RB, 128, 128), lambda i: (i, 0, 0)),
        out_shape=jax.ShapeDtypeStruct((ROWS, 128, 128), jnp.float32),
    )(g, msg_prev, Ws_s, Tb, Ws_rbf, Ws_vn, bs, Wh15, Wv15, Wg, bg, E43)


def _update_call(p, state):
    def body(p_ref, st_ref, o_ref):
        aggs = jnp.concatenate(
            [_unpack_partials(p_ref[0]), _unpack_partials(p_ref[1])], axis=1)
        aggv = _unpack_partials(p_ref[2])
        den = jnp.maximum(aggv[:, 12:13], 1.0)
        s1 = st_ref[:, 0:32] + aggs / den
        v1 = st_ref[:, 32:44] + aggv[:, 0:12] / den
        o_ref[...] = jnp.concatenate(
            [s1, v1, st_ref[:, 44:47], jnp.zeros((NB, 81), jnp.float32)],
            axis=1)

    return pl.pallas_call(
        body,
        grid=(NP // NB,),
        in_specs=[pl.BlockSpec((3, 2, PRS, 128), lambda i: (0, 0, i, 0)),
                  pl.BlockSpec((NB, 128), lambda i: (i, 0))],
        out_specs=pl.BlockSpec((NB, 128), lambda i: (i, 0)),
        out_shape=jax.ShapeDtypeStruct((NP, 128), jnp.float32),
    )(p, state)


def _mlp_call(state, W_l1, b_l1, W_l2, b_l2):
    def body(st_ref, w1_ref, b1_ref, w2_ref, b2_ref, o_ref):
        h = jnp.maximum(st_ref[:, 0:32] @ w1_ref[...] + b1_ref[...], 0.0)
        o_ref[...] = h @ w2_ref[...] + b2_ref[...]

    return pl.pallas_call(
        body,
        grid=(NP // NB,),
        in_specs=[pl.BlockSpec((NB, 128), lambda i: (i, 0)),
                  pl.BlockSpec((32, 16), lambda i: (0, 0)),
                  pl.BlockSpec((1, 16), lambda i: (0, 0)),
                  pl.BlockSpec((16, 8), lambda i: (0, 0)),
                  pl.BlockSpec((1, 8), lambda i: (0, 0))],
        out_specs=pl.BlockSpec((NB, LATENT), lambda i: (i, 0)),
        out_shape=jax.ShapeDtypeStruct((NP, LATENT), jnp.float32),
    )(state, W_l1, b_l1, W_l2, b_l2)


def _layer_weights(Wh, Ws, bs, Wv, Wg, bg, e_table, layer0):
    Ws_s = Ws[0:32]
    Tb = e_table @ Ws[32:40]
    Ws_rbf = Ws[40:50]
    Ws_vn = Ws[50:55]
    perm = jnp.array([3 * h + c for c in range(3) for h in range(5)],
                     dtype=jnp.int32)
    if layer0:
        Wmat = jnp.kron(Wh[4:5, :], jnp.eye(3, dtype=jnp.float32))
    else:
        Wmat = jnp.kron(Wh, jnp.eye(3, dtype=jnp.float32))
    Wmat = Wmat[:, perm]
    Wv15 = jnp.kron(Wv, jnp.eye(3, dtype=jnp.float32))[perm, :]
    E43 = jnp.kron(jnp.eye(4, dtype=jnp.float32), jnp.ones((1, 3), jnp.float32))
    return (Ws_s, Tb, Ws_rbf, Ws_vn, bs[None, :], Wmat, Wv15, Wg, bg[None, :], E43)


def kernel(atom_types, atom_charges, bond_orders, coords, edge_index, a_table, c_table, e_table, W_ns, b_ns, Wh_0, Ws_0, bs_0, Wv_0, Wg_0, bg_0, Wh_1, Ws_1, bs_1, Wv_1, Wg_1, bg_1, Wh_2, Ws_2, bs_2, Wv_2, Wg_2, bg_2, W_l1, b_l1, W_l2, b_l2):
    pad = EP - E
    pad_idx = (jnp.arange(pad, dtype=jnp.int32) * 61) % N
    src_r = jnp.concatenate([edge_index[0].astype(jnp.int32), pad_idx]).reshape(ROWS, 128)
    dst_r = jnp.concatenate([edge_index[1].astype(jnp.int32), pad_idx]).reshape(ROWS, 128)
    bond_r = jnp.concatenate([bond_orders.astype(jnp.float32),
                              jnp.zeros((pad,), jnp.float32)]).reshape(ROWS, 128)
    feat = jnp.concatenate(
        [atom_types.astype(jnp.float32)[:, None],
         atom_charges.astype(jnp.float32)[:, None], coords,
         jnp.zeros((N, 3), jnp.float32)], axis=1)
    feat = jnp.concatenate([feat, jnp.zeros((NP - N, 8), jnp.float32)], axis=0)
    A2 = a_table @ W_ns[0:16]
    C2 = c_table @ W_ns[16:24]
    w0 = _layer_weights(Wh_0, Ws_0, bs_0, Wv_0, Wg_0, bg_0, e_table, False)
    w1 = _layer_weights(Wh_1, Ws_1, bs_1, Wv_1, Wg_1, bg_1, e_table, False)
    w2 = _layer_weights(Wh_2, Ws_2, bs_2, Wv_2, Wg_2, bg_2, e_table, False)

    state0 = _embed_call(feat, A2, C2, b_ns[None, :])

    ctab = state0[:, 44:60]
    gs0 = _sc_gather(ctab, src_r, 16)
    gd0 = _sc_gather(ctab, dst_r, 16)
    msg_init = _geom_call(gs0, gd0, bond_r)

    state, msg = state0, msg_init
    for wl in (w0, w1, w2):
        g = _sc_gather(state[:, 0:64], src_r, 64)
        msg = _edgeN_call(g, msg, wl)
        p = _sc_scatter(msg, dst_r, 3)
        state = _update_call(p, state)
    state3 = state

    lat = _mlp_call(state3, W_l1, b_l1[None, :], W_l2, b_l2[None, :])

    atom_latents = lat[:N]
    mask = jnp.zeros((N,), dtype=bool)
    return (atom_latents, mask)
